# baseline (speedup 1.0000x reference)
import jax
import jax.numpy as jnp
from jax.experimental import pallas as pl

N_GRAPHS = 256
EPS = 1e-5


def _gcn(x, edge_index, W, b):
    N = x.shape[0]
    sl = jnp.arange(N, dtype=edge_index.dtype)
    src = jnp.concatenate([edge_index[0], sl])
    dst = jnp.concatenate([edge_index[1], sl])
    h = x @ W
    deg = jnp.zeros((N,), dtype=x.dtype).at[dst].add(1.0)
    dinv = jax.lax.rsqrt(deg)
    norm = dinv[src] * dinv[dst]
    out = jnp.zeros((N, W.shape[1]), dtype=x.dtype).at[dst].add(h[src] * norm[:, None])
    return out + b


def _bn(x, g, be):
    mu = jnp.mean(x, axis=0)
    var = jnp.mean((x - mu) ** 2, axis=0)
    return g * (x - mu) * jax.lax.rsqrt(var + EPS) + be


def _div_kernel(s_ref, c_ref, o_ref):
    o_ref[...] = s_ref[...] / jnp.maximum(c_ref[...], 1.0)


def kernel(x, edge_index, batch, W1, b1, W2, b2, W3, b3, g1, be1, g2, be2, g3, be3):
    h = jax.nn.relu(_bn(_gcn(x, edge_index, W1, b1), g1, be1))
    h = jax.nn.relu(_bn(_gcn(h, edge_index, W2, b2), g2, be2))
    h = jax.nn.relu(_bn(_gcn(h, edge_index, W3, b3), g3, be3))
    sums = jnp.zeros((N_GRAPHS, h.shape[1]), dtype=h.dtype).at[batch].add(h)
    cnt = jnp.zeros((N_GRAPHS,), dtype=h.dtype).at[batch].add(1.0)
    cnt2 = jnp.broadcast_to(cnt[:, None], sums.shape)
    return pl.pallas_call(
        _div_kernel,
        out_shape=jax.ShapeDtypeStruct(sums.shape, sums.dtype),
    )(sums, cnt2)


# speedup vs baseline: 14.3553x; 14.3553x over previous
import functools

import jax
import jax.numpy as jnp
from jax import lax
from jax.experimental import pallas as pl
from jax.experimental.pallas import tpu as pltpu
from jax.experimental.pallas import tpu_sc as plsc

N = 50000
N_PAD = 50048
E = 800000
G = 256
EPS = 1e-5
CW = 32
EG = 128
TPS = 16
ROWS_PER_TILE = N_PAD // TPS
CP = 184
NB = 8
GPT = 392
NBLK_E = GPT // NB
NEG_PAD = GPT * TPS
E_PAD = NEG_PAD * EG
PAD_SRC = 50016
PAD_DST = 50040
RB = 1000
NBLK = N // RB

_MESH = plsc.VectorSubcoreMesh(core_axis_name="c", subcore_axis_name="s")
_SC_PARAMS = pltpu.CompilerParams(use_tc_tiling_on_sc=False)


def _zero_vmem(ref, nrows, width):
    z = jnp.zeros((16,), jnp.float32)

    def body(i, _):
        for c in range(width // 16):
            ref[i, pl.ds(c * 16, 16)] = z
        return 0

    lax.fori_loop(0, nrows, body, 0)


def _fill_ones(ref, nrows, width):
    o = jnp.ones((16,), jnp.float32)

    def body(i, _):
        for c in range(width // 16):
            ref[i, pl.ds(c * 16, 16)] = o
        return 0

    lax.fori_loop(0, nrows, body, 0)


def _make_agg(nchunks):
    npass = nchunks // 2
    out_type = [jax.ShapeDtypeStruct((N_PAD, CW), jnp.float32) for _ in range(nchunks)]
    scratch = [
        pltpu.VMEM_SHARED((N_PAD, CW), jnp.float32),
        pltpu.VMEM((2 * NB, EG), jnp.int32),
        pltpu.VMEM((2 * NB, EG), jnp.int32),
        pltpu.VMEM((4, EG, CW), jnp.float32),
        pltpu.VMEM((CP, CW), jnp.float32),
        pltpu.SemaphoreType.DMA,
        pltpu.SemaphoreType.DMA,
    ]

    def body(srcg, dstg, *rest):
        hs = rest[:nchunks]
        outs = rest[nchunks:2 * nchunks]
        acc, sbuf, dbuf, rows, stage, sem_i, sem_g = rest[2 * nchunks:]
        cid = lax.axis_index("c")
        tid = lax.axis_index("s")
        rbase = tid * ROWS_PER_TILE

        def edge_loop(hs_c):
            g0 = tid * GPT
            pltpu.async_copy(srcg.at[pl.ds(g0, NB), :], sbuf.at[pl.ds(0, NB), :], sem_i)
            pltpu.async_copy(dstg.at[pl.ds(g0, NB), :], dbuf.at[pl.ds(0, NB), :], sem_i)

            def blk(b, _):
                s = lax.rem(b, 2) * NB
                pltpu.make_async_copy(srcg.at[pl.ds(g0, NB), :],
                                      sbuf.at[pl.ds(0, NB), :], sem_i).wait()
                pltpu.make_async_copy(dstg.at[pl.ds(g0, NB), :],
                                      dbuf.at[pl.ds(0, NB), :], sem_i).wait()

                @pl.when(b + 1 < NBLK_E)
                def _():
                    gn = g0 + (b + 1) * NB
                    sn = (NB - s)
                    pltpu.async_copy(srcg.at[pl.ds(gn, NB), :],
                                     sbuf.at[pl.ds(sn, NB), :], sem_i)
                    pltpu.async_copy(dstg.at[pl.ds(gn, NB), :],
                                     dbuf.at[pl.ds(sn, NB), :], sem_i)

                for j in range(4):
                    pltpu.async_copy(hs_c.at[sbuf.at[s + j]], rows.at[j], sem_g)
                for j in range(NB):
                    pltpu.make_async_copy(hs_c.at[sbuf.at[s + j]],
                                          rows.at[j % 4], sem_g).wait()
                    pltpu.sync_copy(rows.at[j % 4], acc.at[dbuf.at[s + j]], add=True)
                    if j + 4 < NB:
                        pltpu.async_copy(hs_c.at[sbuf.at[s + j + 4]],
                                         rows.at[j % 4], sem_g)
                return 0

            lax.fori_loop(0, NBLK_E, blk, 0)

        for p in range(npass):
            _zero_vmem(stage, CP, CW)
            for j in range(ROWS_PER_TILE // CP):
                pltpu.sync_copy(stage, acc.at[pl.ds(rbase + j * CP, CP), :])
            plsc.subcore_barrier()
            for sc in range(2):
                chunk = sc * npass + p

                @pl.when(cid == sc)
                def _(chunk=chunk):
                    edge_loop(hs[chunk])

            plsc.subcore_barrier()
            for sc in range(2):
                chunk = sc * npass + p

                @pl.when(cid == sc)
                def _(chunk=chunk):
                    for j in range(ROWS_PER_TILE // CP):
                        sl = pl.ds(rbase + j * CP, CP)
                        pltpu.sync_copy(acc.at[sl, :], stage)
                        pltpu.sync_copy(stage, outs[chunk].at[sl, :])

            if p + 1 < npass:
                plsc.subcore_barrier()

    return pl.kernel(body, out_type=out_type, mesh=_MESH, scratch_types=scratch,
                     compiler_params=_SC_PARAMS)


_DEGW = 16
_DGRP_PER_SC = NEG_PAD // 2
_DGRP_PER_TILE = _DGRP_PER_SC // TPS
_DNB = 28


def _deg_body(dstg, out0, out1, dacc, didx, ones, zstage, sem_i, sem_s):
    cid = lax.axis_index("c")
    tid = lax.axis_index("s")
    rbase = tid * ROWS_PER_TILE
    g0 = cid * _DGRP_PER_SC + tid * _DGRP_PER_TILE

    _zero_vmem(zstage, CP, _DEGW)
    _fill_ones(ones, EG, _DEGW)
    for j in range(ROWS_PER_TILE // CP):
        pltpu.sync_copy(zstage, dacc.at[pl.ds(rbase + j * CP, CP), :])
    plsc.subcore_barrier()

    nblk = _DGRP_PER_TILE // _DNB
    pltpu.async_copy(dstg.at[pl.ds(g0, _DNB), :],
                     didx.at[pl.ds(0, _DNB), :], sem_i)

    def blk(b, _):
        s = lax.rem(b, 2) * _DNB
        pltpu.make_async_copy(dstg.at[pl.ds(g0, _DNB), :],
                              didx.at[pl.ds(0, _DNB), :], sem_i).wait()

        @pl.when(b + 1 < nblk)
        def _():
            pltpu.async_copy(dstg.at[pl.ds(g0 + (b + 1) * _DNB, _DNB), :],
                             didx.at[pl.ds(_DNB - s, _DNB), :], sem_i)

        for j in range(_DNB):
            pltpu.async_copy(ones, dacc.at[didx.at[s + j]], sem_s, add=True)
        for j in range(_DNB):
            pltpu.make_async_copy(ones, dacc.at[didx.at[s + j]], sem_s).wait()
        return 0

    lax.fori_loop(0, nblk, blk, 0)
    plsc.subcore_barrier()

    for sc, out in ((0, out0), (1, out1)):
        @pl.when(cid == sc)
        def _(out=out):
            for j in range(ROWS_PER_TILE // CP):
                sl = pl.ds(rbase + j * CP, CP)
                pltpu.sync_copy(dacc.at[sl, :], zstage)
                pltpu.sync_copy(zstage, out.at[sl, :])


_deg_kernel = pl.kernel(
    _deg_body,
    out_type=[jax.ShapeDtypeStruct((N_PAD, _DEGW), jnp.float32)] * 2,
    mesh=_MESH,
    scratch_types=[
        pltpu.VMEM_SHARED((N_PAD, _DEGW), jnp.float32),
        pltpu.VMEM((2 * _DNB, EG), jnp.int32),
        pltpu.VMEM((EG, _DEGW), jnp.float32),
        pltpu.VMEM((CP, _DEGW), jnp.float32),
        pltpu.SemaphoreType.DMA,
        pltpu.SemaphoreType.DMA,
    ],
    compiler_params=_SC_PARAMS,
)


_PG = 80
_NPG = N // _PG
_PROWS = G // TPS
_PGMAX = 40


def _pool_body(batchg, h0, h1, h2, h3, s0, s1, s2, s3, cnt_out,
               pacc, cacc, bidx, hrows, ones, pstage, cstage, sem_g, sem_c):
    hs = (h0, h1, h2, h3)
    outs = (s0, s1, s2, s3)
    cid = lax.axis_index("c")
    tid = lax.axis_index("s")
    rbase = tid * _PROWS
    gs = jnp.where(tid == 0, 0, _PGMAX + (tid - 1) * (_PGMAX - 1))
    ngrp = jnp.where(tid == 0, _PGMAX, _PGMAX - 1)

    _zero_vmem(pstage, _PROWS, CW)
    _zero_vmem(cstage, _PROWS, _DEGW)
    _fill_ones(ones, _PG, _DEGW)
    pltpu.sync_copy(batchg.at[pl.ds(gs, _PGMAX), :], bidx)

    def sum_loop(h_c, count_too):
        pltpu.async_copy(h_c.at[pl.ds(gs * _PG, _PG), :], hrows.at[0], sem_g)

        def step(g, _):
            slot = lax.rem(g, 2)
            pltpu.make_async_copy(h_c.at[pl.ds(gs * _PG, _PG), :],
                                  hrows.at[slot], sem_g).wait()

            @pl.when(g + 1 < ngrp)
            def _():
                pltpu.async_copy(h_c.at[pl.ds((gs + g + 1) * _PG, _PG), :],
                                 hrows.at[1 - slot], sem_g)

            pltpu.sync_copy(hrows.at[slot], pacc.at[bidx.at[g]], add=True)
            if count_too:
                pltpu.async_copy(ones, cacc.at[bidx.at[g]], sem_c, add=True)
            return 0

        lax.fori_loop(0, ngrp, step, 0)
        if count_too:
            def drain(g, _):
                pltpu.make_async_copy(ones, cacc.at[bidx.at[0]], sem_c).wait()
                return 0

            lax.fori_loop(0, ngrp, drain, 0)

    for p in range(2):
        pltpu.sync_copy(pstage, pacc.at[pl.ds(rbase, _PROWS), :])
        if p == 0:
            pltpu.sync_copy(cstage, cacc.at[pl.ds(rbase, _PROWS), :])
        plsc.subcore_barrier()
        for sc in range(2):
            chunk = sc * 2 + p

            @pl.when(cid == sc)
            def _(chunk=chunk, sc=sc):
                sum_loop(hs[chunk], count_too=(p == 0 and sc == 0))

        plsc.subcore_barrier()
        for sc in range(2):
            chunk = sc * 2 + p

            @pl.when(cid == sc)
            def _(chunk=chunk):
                sl = pl.ds(rbase, _PROWS)
                pltpu.sync_copy(pacc.at[sl, :], pstage)
                pltpu.sync_copy(pstage, outs[chunk].at[sl, :])
                _zero_vmem(pstage, _PROWS, CW)

        if p == 0:
            @pl.when(cid == 0)
            def _():
                sl = pl.ds(rbase, _PROWS)
                pltpu.sync_copy(cacc.at[sl, :], cstage)
                pltpu.sync_copy(cstage, cnt_out.at[sl, :])

        if p == 0:
            plsc.subcore_barrier()


_pool_kernel = pl.kernel(
    _pool_body,
    out_type=[jax.ShapeDtypeStruct((G, CW), jnp.float32)] * 4
    + [jax.ShapeDtypeStruct((G, _DEGW), jnp.float32)],
    mesh=_MESH,
    scratch_types=[
        pltpu.VMEM_SHARED((G, CW), jnp.float32),
        pltpu.VMEM_SHARED((G, _DEGW), jnp.float32),
        pltpu.VMEM((_PGMAX, _PG), jnp.int32),
        pltpu.VMEM((2, _PG, CW), jnp.float32),
        pltpu.VMEM((_PG, _DEGW), jnp.float32),
        pltpu.VMEM((_PROWS, CW), jnp.float32),
        pltpu.VMEM((_PROWS, _DEGW), jnp.float32),
        pltpu.SemaphoreType.DMA,
        pltpu.SemaphoreType.DMA,
    ],
    compiler_params=_SC_PARAMS,
)


def _mm_body(h_ref, w_ref, d0_ref, d1_ref, *o_refs):
    dinv = lax.rsqrt(d0_ref[:, :1] + d1_ref[:, :1] + 1.0)
    prod = jnp.dot(h_ref[...], w_ref[...],
                   preferred_element_type=jnp.float32,
                   precision=lax.Precision.HIGHEST)
    prod = prod * dinv
    for c, o in enumerate(o_refs):
        o[...] = prod[:, c * CW:(c + 1) * CW]


def _mm_kernel(h, w, d0, d1):
    fi, fo = w.shape
    nc = fo // CW
    return pl.pallas_call(
        _mm_body,
        grid=(NBLK,),
        in_specs=[
            pl.BlockSpec((RB, fi), lambda i: (i, 0)),
            pl.BlockSpec((fi, fo), lambda i: (0, 0)),
            pl.BlockSpec((RB, _DEGW), lambda i: (i, 0)),
            pl.BlockSpec((RB, _DEGW), lambda i: (i, 0)),
        ],
        out_specs=[pl.BlockSpec((RB, CW), lambda i: (i, 0))] * nc,
        out_shape=[jax.ShapeDtypeStruct((N_PAD, CW), jnp.float32)] * nc,
    )(h, w, d0, d1)


def _fused_body(nc_in, nc_out, has_mm, *refs):
    y = refs[:nc_in]
    hsc = refs[nc_in:2 * nc_in]
    pos = 2 * nc_in
    b_ref, g_ref, be_ref, d0_ref, d1_ref = refs[pos:pos + 5]
    pos += 5
    if has_mm:
        w_ref = refs[pos]
        pos += 1
    o_refs = refs[pos:pos + nc_out]
    t_buf, acc = refs[pos + nc_out:]
    p = pl.program_id(0)
    i = pl.program_id(1)
    dinv = lax.rsqrt(d0_ref[:, :1] + d1_ref[:, :1] + 1.0)

    @pl.when(jnp.logical_and(p == 0, i == 0))
    def _():
        acc[...] = jnp.zeros_like(acc)

    @pl.when(p == 0)
    def _():
        yf = jnp.concatenate([r[...] for r in y], axis=1)
        hf = jnp.concatenate([r[...] for r in hsc], axis=1)
        t = dinv * (yf + hf) + b_ref[...]
        t_buf[pl.ds(i * RB, RB), :] = t
        acc[0:1, :] += jnp.sum(t, axis=0, keepdims=True)
        acc[1:2, :] += jnp.sum(t * t, axis=0, keepdims=True)

    @pl.when(p == 1)
    def _():
        t = t_buf[pl.ds(i * RB, RB), :]
        mu = acc[0:1, :] / N
        var = acc[1:2, :] / N - mu * mu
        hn = g_ref[...] * (t - mu) * lax.rsqrt(var + EPS) + be_ref[...]
        hn = jnp.maximum(hn, 0.0)
        if has_mm:
            prod = jnp.dot(hn, w_ref[...],
                           preferred_element_type=jnp.float32,
                           precision=lax.Precision.HIGHEST)
            prod = prod * dinv
        else:
            prod = hn
        for c, o in enumerate(o_refs):
            o[...] = prod[:, c * CW:(c + 1) * CW]


def _fused_kernel(y_chunks, hs_chunks, b, g, be, d0, d1, w):
    nc_in = len(y_chunks)
    fi = nc_in * CW
    has_mm = w is not None
    fo = w.shape[1] if has_mm else fi
    nc_out = fo // CW
    body = functools.partial(_fused_body, nc_in, nc_out, has_mm)
    row = lambda p, i: (i, 0)
    phase0_row = lambda p, i: (i * (1 - p), 0)
    const = lambda p, i: (0, 0)
    in_specs = (
        [pl.BlockSpec((RB, CW), phase0_row)] * (2 * nc_in)
        + [pl.BlockSpec((1, fi), const),
           pl.BlockSpec((1, fi), const),
           pl.BlockSpec((1, fi), const),
           pl.BlockSpec((RB, _DEGW), row),
           pl.BlockSpec((RB, _DEGW), row)]
    )
    args = list(y_chunks) + list(hs_chunks) + [
        b.reshape(1, fi), g.reshape(1, fi), be.reshape(1, fi), d0, d1]
    if has_mm:
        in_specs.append(pl.BlockSpec((fi, fo), const))
        args.append(w)
    out_rows = N_PAD if has_mm else N
    return pl.pallas_call(
        body,
        grid=(2, NBLK),
        in_specs=in_specs,
        out_specs=[pl.BlockSpec((RB, CW), lambda p, i: (i * p, 0))] * nc_out,
        out_shape=[jax.ShapeDtypeStruct((out_rows, CW), jnp.float32)] * nc_out,
        scratch_shapes=[pltpu.VMEM((N, fi), jnp.float32),
                        pltpu.VMEM((8, fi), jnp.float32)],
    )(*args)


def _final_body(p0, p1, p2, p3, cnt, o_ref):
    sums = jnp.concatenate([p0[...], p1[...], p2[...], p3[...]], axis=1)
    c = jnp.maximum(cnt[:, :1], 1.0)
    o_ref[...] = sums / c


def _final_kernel(pc, cnt):
    return pl.pallas_call(
        _final_body,
        in_specs=[pl.BlockSpec((G, CW), lambda: (0, 0))] * 4
        + [pl.BlockSpec((G, _DEGW), lambda: (0, 0))],
        out_specs=pl.BlockSpec((G, 4 * CW), lambda: (0, 0)),
        out_shape=jax.ShapeDtypeStruct((G, 4 * CW), jnp.float32),
    )(*pc, cnt)


_agg2 = _make_agg(2)
_agg4 = _make_agg(4)


def kernel(x, edge_index, batch, W1, b1, W2, b2, W3, b3, g1, be1, g2, be2, g3, be3):
    npad = E_PAD - E
    srcg = jnp.concatenate(
        [edge_index[0], jnp.full((npad,), PAD_SRC, jnp.int32)]).reshape(NEG_PAD, EG)
    dstg = jnp.concatenate(
        [edge_index[1], jnp.full((npad,), PAD_DST, jnp.int32)]).reshape(NEG_PAD, EG)
    batchg = jnp.concatenate(
        [batch, jnp.zeros((632 * _PG - N,), jnp.int32)]).reshape(632, _PG)

    d0, d1 = _deg_kernel(dstg)

    hs1 = _mm_kernel(x, W1, d0, d1)
    y1 = _agg2(srcg, dstg, *hs1)
    hs2 = _fused_kernel(y1, hs1, b1, g1, be1, d0, d1, W2)
    y2 = _agg4(srcg, dstg, *hs2)
    hs3 = _fused_kernel(y2, hs2, b2, g2, be2, d0, d1, W3)
    y3 = _agg4(srcg, dstg, *hs3)
    h = _fused_kernel(y3, hs3, b3, g3, be3, d0, d1, None)
    *pc, cnt = _pool_kernel(batchg, *h)
    return _final_kernel(pc, cnt)


# speedup vs baseline: 15.0771x; 1.0503x over previous
import functools

import jax
import jax.numpy as jnp
from jax import lax
from jax.experimental import pallas as pl
from jax.experimental.pallas import tpu as pltpu
from jax.experimental.pallas import tpu_sc as plsc

N = 50000
N_PAD = 50048
E = 800000
G = 256
EPS = 1e-5
CW = 32
EG = 128
TPS = 16
ROWS_PER_TILE = N_PAD // TPS
CP = 184
NB = 8
GPT = 392
NBLK_E = GPT // NB
NEG_PAD = GPT * TPS
E_PAD = NEG_PAD * EG
PAD_SRC = 50016
PAD_DST = 50040
RB = 1000
NBLK = N // RB

_MESH = plsc.VectorSubcoreMesh(core_axis_name="c", subcore_axis_name="s")
_SC_PARAMS = pltpu.CompilerParams(use_tc_tiling_on_sc=False)


def _zero_vmem(ref, nrows, width):
    z = jnp.zeros((16,), jnp.float32)

    def body(i, _):
        for c in range(width // 16):
            ref[i, pl.ds(c * 16, 16)] = z
        return 0

    lax.fori_loop(0, nrows, body, 0)


def _fill_ones(ref, nrows, width):
    o = jnp.ones((16,), jnp.float32)

    def body(i, _):
        for c in range(width // 16):
            ref[i, pl.ds(c * 16, 16)] = o
        return 0

    lax.fori_loop(0, nrows, body, 0)


def _make_agg(nchunks):
    npass = nchunks // 2
    out_type = [jax.ShapeDtypeStruct((N_PAD, CW), jnp.float32) for _ in range(nchunks)]
    scratch = [
        pltpu.VMEM_SHARED((N_PAD, CW), jnp.float32),
        pltpu.VMEM((3 * NB, EG), jnp.int32),
        pltpu.VMEM((3 * NB, EG), jnp.int32),
        pltpu.VMEM((5, EG, CW), jnp.float32),
        pltpu.VMEM((EG, CW), jnp.float32),
        pltpu.SemaphoreType.DMA,
        pltpu.SemaphoreType.DMA,
        pltpu.SemaphoreType.DMA,
    ]
    ncp = ROWS_PER_TILE // EG
    tail = ROWS_PER_TILE - ncp * EG

    def body(srcg, dstg, *rest):
        hs = rest[:nchunks]
        outs = rest[nchunks:2 * nchunks]
        acc, sbuf, dbuf, rows, stage, sem_i, sem_g, sem_s = rest[2 * nchunks:]
        cid = lax.axis_index("c")
        tid = lax.axis_index("s")
        rbase = tid * ROWS_PER_TILE

        def edge_loop(hs_c):
            g0 = tid * GPT
            pltpu.sync_copy(srcg.at[pl.ds(g0, NB), :], sbuf.at[pl.ds(0, NB), :])
            pltpu.sync_copy(dstg.at[pl.ds(g0, NB), :], dbuf.at[pl.ds(0, NB), :])
            pltpu.async_copy(srcg.at[pl.ds(g0 + NB, NB), :],
                             sbuf.at[pl.ds(NB, NB), :], sem_i)
            pltpu.async_copy(dstg.at[pl.ds(g0 + NB, NB), :],
                             dbuf.at[pl.ds(NB, NB), :], sem_i)
            for j in range(3):
                pltpu.async_copy(hs_c.at[sbuf.at[j]], rows.at[j], sem_g)

            def blk(b, _):
                s = lax.rem(b, 3) * NB
                for j in range(NB):
                    gg = b * NB + j
                    slot = lax.rem(gg, 5)
                    pltpu.make_async_copy(hs_c.at[sbuf.at[s + j]],
                                          rows.at[slot], sem_g).wait()

                    @pl.when(gg >= 2)
                    def _():
                        pltpu.make_async_copy(rows.at[0], acc.at[dbuf.at[0]],
                                              sem_s).wait()

                    pltpu.async_copy(rows.at[slot], acc.at[dbuf.at[s + j]],
                                     sem_s, add=True)
                    if j == 5:
                        @pl.when(b + 1 < NBLK_E)
                        def _():
                            pltpu.make_async_copy(
                                srcg.at[pl.ds(g0, NB), :],
                                sbuf.at[pl.ds(0, NB), :], sem_i).wait()
                            pltpu.make_async_copy(
                                dstg.at[pl.ds(g0, NB), :],
                                dbuf.at[pl.ds(0, NB), :], sem_i).wait()

                        @pl.when(b + 2 < NBLK_E)
                        def _():
                            gn = g0 + (b + 2) * NB
                            s2 = lax.rem(b + 2, 3) * NB
                            pltpu.async_copy(srcg.at[pl.ds(gn, NB), :],
                                             sbuf.at[pl.ds(s2, NB), :], sem_i)
                            pltpu.async_copy(dstg.at[pl.ds(gn, NB), :],
                                             dbuf.at[pl.ds(s2, NB), :], sem_i)

                    if j < NB - 3:
                        sb_row = s + j + 3
                    else:
                        sb_row = lax.rem(b + 1, 3) * NB + (j + 3 - NB)
                    gslot = lax.rem(gg + 3, 5)

                    @pl.when(gg + 3 < GPT)
                    def _(sb_row=sb_row, gslot=gslot):
                        pltpu.async_copy(hs_c.at[sbuf.at[sb_row]],
                                         rows.at[gslot], sem_g)
                return 0

            lax.fori_loop(0, NBLK_E, blk, 0)
            pltpu.make_async_copy(rows.at[0], acc.at[dbuf.at[0]], sem_s).wait()
            pltpu.make_async_copy(rows.at[0], acc.at[dbuf.at[0]], sem_s).wait()

        for p in range(npass):
            _zero_vmem(stage, EG, CW)
            for j in range(ncp):
                pltpu.sync_copy(stage, acc.at[pl.ds(rbase + j * EG, EG), :])
            pltpu.sync_copy(stage.at[pl.ds(0, tail), :],
                            acc.at[pl.ds(rbase + ncp * EG, tail), :])
            plsc.subcore_barrier()
            for sc in range(2):
                chunk = sc * npass + p

                @pl.when(cid == sc)
                def _(chunk=chunk):
                    edge_loop(hs[chunk])

            plsc.subcore_barrier()
            for sc in range(2):
                chunk = sc * npass + p

                @pl.when(cid == sc)
                def _(chunk=chunk):
                    for j in range(ncp):
                        sl = pl.ds(rbase + j * EG, EG)
                        pltpu.sync_copy(acc.at[sl, :], stage)
                        pltpu.sync_copy(stage, outs[chunk].at[sl, :])
                    sl = pl.ds(rbase + ncp * EG, tail)
                    pltpu.sync_copy(acc.at[sl, :], stage.at[pl.ds(0, tail), :])
                    pltpu.sync_copy(stage.at[pl.ds(0, tail), :],
                                    outs[chunk].at[sl, :])

            if p + 1 < npass:
                plsc.subcore_barrier()

    return pl.kernel(body, out_type=out_type, mesh=_MESH, scratch_types=scratch,
                     compiler_params=_SC_PARAMS)


_DEGW = 16
_DGRP_PER_SC = NEG_PAD // 2
_DGRP_PER_TILE = _DGRP_PER_SC // TPS
_DNB = 28


def _deg_body(dstg, out0, out1, dacc, didx, ones, zstage, sem_i, sem_s):
    cid = lax.axis_index("c")
    tid = lax.axis_index("s")
    rbase = tid * ROWS_PER_TILE
    g0 = cid * _DGRP_PER_SC + tid * _DGRP_PER_TILE

    _zero_vmem(zstage, CP, _DEGW)
    _fill_ones(ones, EG, _DEGW)
    for j in range(ROWS_PER_TILE // CP):
        pltpu.sync_copy(zstage, dacc.at[pl.ds(rbase + j * CP, CP), :])
    plsc.subcore_barrier()

    nblk = _DGRP_PER_TILE // _DNB
    pltpu.async_copy(dstg.at[pl.ds(g0, _DNB), :],
                     didx.at[pl.ds(0, _DNB), :], sem_i)

    def blk(b, _):
        s = lax.rem(b, 2) * _DNB
        pltpu.make_async_copy(dstg.at[pl.ds(g0, _DNB), :],
                              didx.at[pl.ds(0, _DNB), :], sem_i).wait()

        @pl.when(b + 1 < nblk)
        def _():
            pltpu.async_copy(dstg.at[pl.ds(g0 + (b + 1) * _DNB, _DNB), :],
                             didx.at[pl.ds(_DNB - s, _DNB), :], sem_i)

        for j in range(_DNB):
            pltpu.async_copy(ones, dacc.at[didx.at[s + j]], sem_s, add=True)
        for j in range(_DNB):
            pltpu.make_async_copy(ones, dacc.at[didx.at[s + j]], sem_s).wait()
        return 0

    lax.fori_loop(0, nblk, blk, 0)
    plsc.subcore_barrier()

    for sc, out in ((0, out0), (1, out1)):
        @pl.when(cid == sc)
        def _(out=out):
            for j in range(ROWS_PER_TILE // CP):
                sl = pl.ds(rbase + j * CP, CP)
                pltpu.sync_copy(dacc.at[sl, :], zstage)
                pltpu.sync_copy(zstage, out.at[sl, :])


_deg_kernel = pl.kernel(
    _deg_body,
    out_type=[jax.ShapeDtypeStruct((N_PAD, _DEGW), jnp.float32)] * 2,
    mesh=_MESH,
    scratch_types=[
        pltpu.VMEM_SHARED((N_PAD, _DEGW), jnp.float32),
        pltpu.VMEM((2 * _DNB, EG), jnp.int32),
        pltpu.VMEM((EG, _DEGW), jnp.float32),
        pltpu.VMEM((CP, _DEGW), jnp.float32),
        pltpu.SemaphoreType.DMA,
        pltpu.SemaphoreType.DMA,
    ],
    compiler_params=_SC_PARAMS,
)


_PG = 80
_NPG = N // _PG
_PROWS = G // TPS
_PGMAX = 40


def _pool_body(batchg, h0, h1, h2, h3, s0, s1, s2, s3, cnt_out,
               pacc, cacc, bidx, hrows, ones, pstage, cstage, sem_g, sem_c):
    hs = (h0, h1, h2, h3)
    outs = (s0, s1, s2, s3)
    cid = lax.axis_index("c")
    tid = lax.axis_index("s")
    rbase = tid * _PROWS
    gs = jnp.where(tid == 0, 0, _PGMAX + (tid - 1) * (_PGMAX - 1))
    ngrp = jnp.where(tid == 0, _PGMAX, _PGMAX - 1)

    _zero_vmem(pstage, _PROWS, CW)
    _zero_vmem(cstage, _PROWS, _DEGW)
    _fill_ones(ones, _PG, _DEGW)
    pltpu.sync_copy(batchg.at[pl.ds(gs, _PGMAX), :], bidx)

    def sum_loop(h_c, count_too):
        pltpu.async_copy(h_c.at[pl.ds(gs * _PG, _PG), :], hrows.at[0], sem_g)

        def step(g, _):
            slot = lax.rem(g, 2)
            pltpu.make_async_copy(h_c.at[pl.ds(gs * _PG, _PG), :],
                                  hrows.at[slot], sem_g).wait()

            @pl.when(g + 1 < ngrp)
            def _():
                pltpu.async_copy(h_c.at[pl.ds((gs + g + 1) * _PG, _PG), :],
                                 hrows.at[1 - slot], sem_g)

            pltpu.sync_copy(hrows.at[slot], pacc.at[bidx.at[g]], add=True)
            if count_too:
                pltpu.async_copy(ones, cacc.at[bidx.at[g]], sem_c, add=True)
            return 0

        lax.fori_loop(0, ngrp, step, 0)
        if count_too:
            def drain(g, _):
                pltpu.make_async_copy(ones, cacc.at[bidx.at[0]], sem_c).wait()
                return 0

            lax.fori_loop(0, ngrp, drain, 0)

    for p in range(2):
        pltpu.sync_copy(pstage, pacc.at[pl.ds(rbase, _PROWS), :])
        if p == 0:
            pltpu.sync_copy(cstage, cacc.at[pl.ds(rbase, _PROWS), :])
        plsc.subcore_barrier()
        for sc in range(2):
            chunk = sc * 2 + p

            @pl.when(cid == sc)
            def _(chunk=chunk, sc=sc):
                sum_loop(hs[chunk], count_too=(p == 0 and sc == 0))

        plsc.subcore_barrier()
        for sc in range(2):
            chunk = sc * 2 + p

            @pl.when(cid == sc)
            def _(chunk=chunk):
                sl = pl.ds(rbase, _PROWS)
                pltpu.sync_copy(pacc.at[sl, :], pstage)
                pltpu.sync_copy(pstage, outs[chunk].at[sl, :])
                _zero_vmem(pstage, _PROWS, CW)

        if p == 0:
            @pl.when(cid == 0)
            def _():
                sl = pl.ds(rbase, _PROWS)
                pltpu.sync_copy(cacc.at[sl, :], cstage)
                pltpu.sync_copy(cstage, cnt_out.at[sl, :])

        if p == 0:
            plsc.subcore_barrier()


_pool_kernel = pl.kernel(
    _pool_body,
    out_type=[jax.ShapeDtypeStruct((G, CW), jnp.float32)] * 4
    + [jax.ShapeDtypeStruct((G, _DEGW), jnp.float32)],
    mesh=_MESH,
    scratch_types=[
        pltpu.VMEM_SHARED((G, CW), jnp.float32),
        pltpu.VMEM_SHARED((G, _DEGW), jnp.float32),
        pltpu.VMEM((_PGMAX, _PG), jnp.int32),
        pltpu.VMEM((2, _PG, CW), jnp.float32),
        pltpu.VMEM((_PG, _DEGW), jnp.float32),
        pltpu.VMEM((_PROWS, CW), jnp.float32),
        pltpu.VMEM((_PROWS, _DEGW), jnp.float32),
        pltpu.SemaphoreType.DMA,
        pltpu.SemaphoreType.DMA,
    ],
    compiler_params=_SC_PARAMS,
)


def _mm_body(h_ref, w_ref, d0_ref, d1_ref, *o_refs):
    dinv = lax.rsqrt(d0_ref[:, :1] + d1_ref[:, :1] + 1.0)
    prod = jnp.dot(h_ref[...], w_ref[...],
                   preferred_element_type=jnp.float32,
                   precision=lax.Precision.HIGHEST)
    prod = prod * dinv
    for c, o in enumerate(o_refs):
        o[...] = prod[:, c * CW:(c + 1) * CW]


def _mm_kernel(h, w, d0, d1):
    fi, fo = w.shape
    nc = fo // CW
    return pl.pallas_call(
        _mm_body,
        grid=(NBLK,),
        in_specs=[
            pl.BlockSpec((RB, fi), lambda i: (i, 0)),
            pl.BlockSpec((fi, fo), lambda i: (0, 0)),
            pl.BlockSpec((RB, _DEGW), lambda i: (i, 0)),
            pl.BlockSpec((RB, _DEGW), lambda i: (i, 0)),
        ],
        out_specs=[pl.BlockSpec((RB, CW), lambda i: (i, 0))] * nc,
        out_shape=[jax.ShapeDtypeStruct((N_PAD, CW), jnp.float32)] * nc,
    )(h, w, d0, d1)


def _fused_body(nc_in, nc_out, has_mm, *refs):
    y = refs[:nc_in]
    hsc = refs[nc_in:2 * nc_in]
    pos = 2 * nc_in
    b_ref, g_ref, be_ref, d0_ref, d1_ref = refs[pos:pos + 5]
    pos += 5
    if has_mm:
        w_ref = refs[pos]
        pos += 1
    o_refs = refs[pos:pos + nc_out]
    t_buf, acc = refs[pos + nc_out:]
    p = pl.program_id(0)
    i = pl.program_id(1)
    dinv = lax.rsqrt(d0_ref[:, :1] + d1_ref[:, :1] + 1.0)

    @pl.when(jnp.logical_and(p == 0, i == 0))
    def _():
        acc[...] = jnp.zeros_like(acc)

    @pl.when(p == 0)
    def _():
        yf = jnp.concatenate([r[...] for r in y], axis=1)
        hf = jnp.concatenate([r[...] for r in hsc], axis=1)
        t = dinv * (yf + hf) + b_ref[...]
        t_buf[pl.ds(i * RB, RB), :] = t
        acc[0:1, :] += jnp.sum(t, axis=0, keepdims=True)
        acc[1:2, :] += jnp.sum(t * t, axis=0, keepdims=True)

    @pl.when(p == 1)
    def _():
        t = t_buf[pl.ds(i * RB, RB), :]
        mu = acc[0:1, :] / N
        var = acc[1:2, :] / N - mu * mu
        hn = g_ref[...] * (t - mu) * lax.rsqrt(var + EPS) + be_ref[...]
        hn = jnp.maximum(hn, 0.0)
        if has_mm:
            prod = jnp.dot(hn, w_ref[...],
                           preferred_element_type=jnp.float32,
                           precision=lax.Precision.HIGHEST)
            prod = prod * dinv
        else:
            prod = hn
        for c, o in enumerate(o_refs):
            o[...] = prod[:, c * CW:(c + 1) * CW]


def _fused_kernel(y_chunks, hs_chunks, b, g, be, d0, d1, w):
    nc_in = len(y_chunks)
    fi = nc_in * CW
    has_mm = w is not None
    fo = w.shape[1] if has_mm else fi
    nc_out = fo // CW
    body = functools.partial(_fused_body, nc_in, nc_out, has_mm)
    row = lambda p, i: (i, 0)
    phase0_row = lambda p, i: (i * (1 - p), 0)
    const = lambda p, i: (0, 0)
    in_specs = (
        [pl.BlockSpec((RB, CW), phase0_row)] * (2 * nc_in)
        + [pl.BlockSpec((1, fi), const),
           pl.BlockSpec((1, fi), const),
           pl.BlockSpec((1, fi), const),
           pl.BlockSpec((RB, _DEGW), row),
           pl.BlockSpec((RB, _DEGW), row)]
    )
    args = list(y_chunks) + list(hs_chunks) + [
        b.reshape(1, fi), g.reshape(1, fi), be.reshape(1, fi), d0, d1]
    if has_mm:
        in_specs.append(pl.BlockSpec((fi, fo), const))
        args.append(w)
    out_rows = N_PAD if has_mm else N
    return pl.pallas_call(
        body,
        grid=(2, NBLK),
        in_specs=in_specs,
        out_specs=[pl.BlockSpec((RB, CW), lambda p, i: (i * p, 0))] * nc_out,
        out_shape=[jax.ShapeDtypeStruct((out_rows, CW), jnp.float32)] * nc_out,
        scratch_shapes=[pltpu.VMEM((N, fi), jnp.float32),
                        pltpu.VMEM((8, fi), jnp.float32)],
    )(*args)


def _final_body(p0, p1, p2, p3, cnt, o_ref):
    sums = jnp.concatenate([p0[...], p1[...], p2[...], p3[...]], axis=1)
    c = jnp.maximum(cnt[:, :1], 1.0)
    o_ref[...] = sums / c


def _final_kernel(pc, cnt):
    return pl.pallas_call(
        _final_body,
        in_specs=[pl.BlockSpec((G, CW), lambda: (0, 0))] * 4
        + [pl.BlockSpec((G, _DEGW), lambda: (0, 0))],
        out_specs=pl.BlockSpec((G, 4 * CW), lambda: (0, 0)),
        out_shape=jax.ShapeDtypeStruct((G, 4 * CW), jnp.float32),
    )(*pc, cnt)


_agg2 = _make_agg(2)
_agg4 = _make_agg(4)


def kernel(x, edge_index, batch, W1, b1, W2, b2, W3, b3, g1, be1, g2, be2, g3, be3):
    npad = E_PAD - E
    srcg = jnp.concatenate(
        [edge_index[0], jnp.full((npad,), PAD_SRC, jnp.int32)]).reshape(NEG_PAD, EG)
    dstg = jnp.concatenate(
        [edge_index[1], jnp.full((npad,), PAD_DST, jnp.int32)]).reshape(NEG_PAD, EG)
    batchg = jnp.concatenate(
        [batch, jnp.zeros((632 * _PG - N,), jnp.int32)]).reshape(632, _PG)

    d0, d1 = _deg_kernel(dstg)

    hs1 = _mm_kernel(x, W1, d0, d1)
    y1 = _agg2(srcg, dstg, *hs1)
    hs2 = _fused_kernel(y1, hs1, b1, g1, be1, d0, d1, W2)
    y2 = _agg4(srcg, dstg, *hs2)
    hs3 = _fused_kernel(y2, hs2, b2, g2, be2, d0, d1, W3)
    y3 = _agg4(srcg, dstg, *hs3)
    h = _fused_kernel(y3, hs3, b3, g3, be3, d0, d1, None)
    *pc, cnt = _pool_kernel(batchg, *h)
    return _final_kernel(pc, cnt)


# speedup vs baseline: 15.3237x; 1.0164x over previous
import functools

import jax
import jax.numpy as jnp
from jax import lax
from jax.experimental import pallas as pl
from jax.experimental.pallas import tpu as pltpu
from jax.experimental.pallas import tpu_sc as plsc

N = 50000
N_PAD = 50048
E = 800000
G = 256
EPS = 1e-5
CW = 32
EG = 128
TPS = 16
ROWS_PER_TILE = N_PAD // TPS
CP = 184
NB = 8
GPT = 392
NBLK_E = GPT // NB
NEG_PAD = GPT * TPS
E_PAD = NEG_PAD * EG
PAD_SRC = 50016
PAD_DST = 50040
RB = 1000
NBLK = N // RB

_MESH = plsc.VectorSubcoreMesh(core_axis_name="c", subcore_axis_name="s")
_SC_PARAMS = pltpu.CompilerParams(use_tc_tiling_on_sc=False)


def _zero_vmem(ref, nrows, width):
    z = jnp.zeros((16,), jnp.float32)

    def body(i, _):
        for c in range(width // 16):
            ref[i, pl.ds(c * 16, 16)] = z
        return 0

    lax.fori_loop(0, nrows, body, 0)


def _fill_ones(ref, nrows, width):
    o = jnp.ones((16,), jnp.float32)

    def body(i, _):
        for c in range(width // 16):
            ref[i, pl.ds(c * 16, 16)] = o
        return 0

    lax.fori_loop(0, nrows, body, 0)


def _make_agg(nchunks):
    npass = nchunks // 2
    out_type = [jax.ShapeDtypeStruct((N_PAD, CW), jnp.float32) for _ in range(nchunks)]
    scratch = [
        pltpu.VMEM_SHARED((N_PAD, CW), jnp.float32),
        pltpu.VMEM((3 * NB, EG), jnp.int32),
        pltpu.VMEM((3 * NB, EG), jnp.int32),
        pltpu.VMEM((5, EG, CW), jnp.float32),
        pltpu.VMEM((EG, CW), jnp.float32),
        pltpu.SemaphoreType.DMA,
        pltpu.SemaphoreType.DMA,
        pltpu.SemaphoreType.DMA,
    ]
    ncp = ROWS_PER_TILE // EG
    tail = ROWS_PER_TILE - ncp * EG

    def body(srcg, dstg, *rest):
        hs = rest[:nchunks]
        outs = rest[nchunks:2 * nchunks]
        acc, sbuf, dbuf, rows, stage, sem_i, sem_g, sem_s = rest[2 * nchunks:]
        cid = lax.axis_index("c")
        tid = lax.axis_index("s")
        rbase = tid * ROWS_PER_TILE

        def edge_loop(hs_c):
            g0 = tid * GPT
            pltpu.sync_copy(srcg.at[pl.ds(g0, NB), :], sbuf.at[pl.ds(0, NB), :])
            pltpu.sync_copy(dstg.at[pl.ds(g0, NB), :], dbuf.at[pl.ds(0, NB), :])
            pltpu.async_copy(srcg.at[pl.ds(g0 + NB, NB), :],
                             sbuf.at[pl.ds(NB, NB), :], sem_i)
            pltpu.async_copy(dstg.at[pl.ds(g0 + NB, NB), :],
                             dbuf.at[pl.ds(NB, NB), :], sem_i)
            for j in range(3):
                pltpu.async_copy(hs_c.at[sbuf.at[j]], rows.at[j], sem_g)

            def blk(b, _):
                s = lax.rem(b, 3) * NB
                for j in range(NB):
                    gg = b * NB + j
                    slot = lax.rem(gg, 5)
                    pltpu.make_async_copy(hs_c.at[sbuf.at[s + j]],
                                          rows.at[slot], sem_g).wait()

                    @pl.when(gg >= 2)
                    def _():
                        pltpu.make_async_copy(rows.at[0], acc.at[dbuf.at[0]],
                                              sem_s).wait()

                    pltpu.async_copy(rows.at[slot], acc.at[dbuf.at[s + j]],
                                     sem_s, add=True)
                    if j == 5:
                        @pl.when(b + 1 < NBLK_E)
                        def _():
                            pltpu.make_async_copy(
                                srcg.at[pl.ds(g0, NB), :],
                                sbuf.at[pl.ds(0, NB), :], sem_i).wait()
                            pltpu.make_async_copy(
                                dstg.at[pl.ds(g0, NB), :],
                                dbuf.at[pl.ds(0, NB), :], sem_i).wait()

                        @pl.when(b + 2 < NBLK_E)
                        def _():
                            gn = g0 + (b + 2) * NB
                            s2 = lax.rem(b + 2, 3) * NB
                            pltpu.async_copy(srcg.at[pl.ds(gn, NB), :],
                                             sbuf.at[pl.ds(s2, NB), :], sem_i)
                            pltpu.async_copy(dstg.at[pl.ds(gn, NB), :],
                                             dbuf.at[pl.ds(s2, NB), :], sem_i)

                    if j < NB - 3:
                        sb_row = s + j + 3
                    else:
                        sb_row = lax.rem(b + 1, 3) * NB + (j + 3 - NB)
                    gslot = lax.rem(gg + 3, 5)

                    @pl.when(gg + 3 < GPT)
                    def _(sb_row=sb_row, gslot=gslot):
                        pltpu.async_copy(hs_c.at[sbuf.at[sb_row]],
                                         rows.at[gslot], sem_g)
                return 0

            lax.fori_loop(0, NBLK_E, blk, 0)
            pltpu.make_async_copy(rows.at[0], acc.at[dbuf.at[0]], sem_s).wait()
            pltpu.make_async_copy(rows.at[0], acc.at[dbuf.at[0]], sem_s).wait()

        for p in range(npass):
            _zero_vmem(stage, EG, CW)
            for j in range(ncp):
                pltpu.sync_copy(stage, acc.at[pl.ds(rbase + j * EG, EG), :])
            pltpu.sync_copy(stage.at[pl.ds(0, tail), :],
                            acc.at[pl.ds(rbase + ncp * EG, tail), :])
            plsc.subcore_barrier()
            for sc in range(2):
                chunk = sc * npass + p

                @pl.when(cid == sc)
                def _(chunk=chunk):
                    edge_loop(hs[chunk])

            plsc.subcore_barrier()
            for sc in range(2):
                chunk = sc * npass + p

                @pl.when(cid == sc)
                def _(chunk=chunk):
                    for j in range(ncp):
                        sl = pl.ds(rbase + j * EG, EG)
                        pltpu.sync_copy(acc.at[sl, :], stage)
                        pltpu.sync_copy(stage, outs[chunk].at[sl, :])
                    sl = pl.ds(rbase + ncp * EG, tail)
                    pltpu.sync_copy(acc.at[sl, :], stage.at[pl.ds(0, tail), :])
                    pltpu.sync_copy(stage.at[pl.ds(0, tail), :],
                                    outs[chunk].at[sl, :])

            if p + 1 < npass:
                plsc.subcore_barrier()

    return pl.kernel(body, out_type=out_type, mesh=_MESH, scratch_types=scratch,
                     compiler_params=_SC_PARAMS)


_DEGW = 16
_DGRP_PER_SC = NEG_PAD // 2
_DGRP_PER_TILE = _DGRP_PER_SC // TPS
_DNB = 28


def _deg_body(dstg, out0, out1, dacc, didx, ones, zstage, sem_i, sem_s):
    cid = lax.axis_index("c")
    tid = lax.axis_index("s")
    rbase = tid * ROWS_PER_TILE
    g0 = cid * _DGRP_PER_SC + tid * _DGRP_PER_TILE

    _zero_vmem(zstage, CP, _DEGW)
    _fill_ones(ones, EG, _DEGW)
    for j in range(ROWS_PER_TILE // CP):
        pltpu.sync_copy(zstage, dacc.at[pl.ds(rbase + j * CP, CP), :])
    plsc.subcore_barrier()

    nblk = _DGRP_PER_TILE // _DNB
    pltpu.async_copy(dstg.at[pl.ds(g0, _DNB), :],
                     didx.at[pl.ds(0, _DNB), :], sem_i)

    def blk(b, _):
        s = lax.rem(b, 2) * _DNB
        pltpu.make_async_copy(dstg.at[pl.ds(g0, _DNB), :],
                              didx.at[pl.ds(0, _DNB), :], sem_i).wait()

        @pl.when(b + 1 < nblk)
        def _():
            pltpu.async_copy(dstg.at[pl.ds(g0 + (b + 1) * _DNB, _DNB), :],
                             didx.at[pl.ds(_DNB - s, _DNB), :], sem_i)

        for j in range(_DNB):
            pltpu.async_copy(ones, dacc.at[didx.at[s + j]], sem_s, add=True)
        for j in range(_DNB):
            pltpu.make_async_copy(ones, dacc.at[didx.at[s + j]], sem_s).wait()
        return 0

    lax.fori_loop(0, nblk, blk, 0)
    plsc.subcore_barrier()

    for sc, out in ((0, out0), (1, out1)):
        @pl.when(cid == sc)
        def _(out=out):
            for j in range(ROWS_PER_TILE // CP):
                sl = pl.ds(rbase + j * CP, CP)
                pltpu.sync_copy(dacc.at[sl, :], zstage)
                pltpu.sync_copy(zstage, out.at[sl, :])


_deg_kernel = pl.kernel(
    _deg_body,
    out_type=[jax.ShapeDtypeStruct((N_PAD, _DEGW), jnp.float32)] * 2,
    mesh=_MESH,
    scratch_types=[
        pltpu.VMEM_SHARED((N_PAD, _DEGW), jnp.float32),
        pltpu.VMEM((2 * _DNB, EG), jnp.int32),
        pltpu.VMEM((EG, _DEGW), jnp.float32),
        pltpu.VMEM((CP, _DEGW), jnp.float32),
        pltpu.SemaphoreType.DMA,
        pltpu.SemaphoreType.DMA,
    ],
    compiler_params=_SC_PARAMS,
)


_PG = 80
_NPG = N // _PG
_PROWS = G // TPS
_PGMAX = 40


def _pool_body(batchg, h0, h1, h2, h3, s0, s1, s2, s3, cnt_out,
               pacc, cacc, bidx, hrows, ones, pstage, cstage, sem_g, sem_c):
    hs = (h0, h1, h2, h3)
    outs = (s0, s1, s2, s3)
    cid = lax.axis_index("c")
    tid = lax.axis_index("s")
    rbase = tid * _PROWS
    gs = jnp.where(tid == 0, 0, _PGMAX + (tid - 1) * (_PGMAX - 1))
    ngrp = jnp.where(tid == 0, _PGMAX, _PGMAX - 1)

    _zero_vmem(pstage, _PROWS, CW)
    _zero_vmem(cstage, _PROWS, _DEGW)
    _fill_ones(ones, _PG, _DEGW)
    pltpu.sync_copy(batchg.at[pl.ds(gs, _PGMAX), :], bidx)

    def sum_loop(h_c, count_too):
        pltpu.async_copy(h_c.at[pl.ds(gs * _PG, _PG), :], hrows.at[0], sem_g)

        def step(g, _):
            slot = lax.rem(g, 2)
            pltpu.make_async_copy(h_c.at[pl.ds(gs * _PG, _PG), :],
                                  hrows.at[slot], sem_g).wait()

            @pl.when(g + 1 < ngrp)
            def _():
                pltpu.async_copy(h_c.at[pl.ds((gs + g + 1) * _PG, _PG), :],
                                 hrows.at[1 - slot], sem_g)

            pltpu.sync_copy(hrows.at[slot], pacc.at[bidx.at[g]], add=True)
            if count_too:
                pltpu.async_copy(ones, cacc.at[bidx.at[g]], sem_c, add=True)
            return 0

        lax.fori_loop(0, ngrp, step, 0)
        if count_too:
            def drain(g, _):
                pltpu.make_async_copy(ones, cacc.at[bidx.at[0]], sem_c).wait()
                return 0

            lax.fori_loop(0, ngrp, drain, 0)

    for p in range(2):
        pltpu.sync_copy(pstage, pacc.at[pl.ds(rbase, _PROWS), :])
        if p == 0:
            pltpu.sync_copy(cstage, cacc.at[pl.ds(rbase, _PROWS), :])
        plsc.subcore_barrier()
        for sc in range(2):
            chunk = sc * 2 + p

            @pl.when(cid == sc)
            def _(chunk=chunk, sc=sc):
                sum_loop(hs[chunk], count_too=(p == 0 and sc == 0))

        plsc.subcore_barrier()
        for sc in range(2):
            chunk = sc * 2 + p

            @pl.when(cid == sc)
            def _(chunk=chunk):
                sl = pl.ds(rbase, _PROWS)
                pltpu.sync_copy(pacc.at[sl, :], pstage)
                pltpu.sync_copy(pstage, outs[chunk].at[sl, :])
                _zero_vmem(pstage, _PROWS, CW)

        if p == 0:
            @pl.when(cid == 0)
            def _():
                sl = pl.ds(rbase, _PROWS)
                pltpu.sync_copy(cacc.at[sl, :], cstage)
                pltpu.sync_copy(cstage, cnt_out.at[sl, :])

        if p == 0:
            plsc.subcore_barrier()


_pool_kernel = pl.kernel(
    _pool_body,
    out_type=[jax.ShapeDtypeStruct((G, CW), jnp.float32)] * 4
    + [jax.ShapeDtypeStruct((G, _DEGW), jnp.float32)],
    mesh=_MESH,
    scratch_types=[
        pltpu.VMEM_SHARED((G, CW), jnp.float32),
        pltpu.VMEM_SHARED((G, _DEGW), jnp.float32),
        pltpu.VMEM((_PGMAX, _PG), jnp.int32),
        pltpu.VMEM((2, _PG, CW), jnp.float32),
        pltpu.VMEM((_PG, _DEGW), jnp.float32),
        pltpu.VMEM((_PROWS, CW), jnp.float32),
        pltpu.VMEM((_PROWS, _DEGW), jnp.float32),
        pltpu.SemaphoreType.DMA,
        pltpu.SemaphoreType.DMA,
    ],
    compiler_params=_SC_PARAMS,
)


def _mm_body(h_ref, w_ref, d0_ref, d1_ref, *o_refs):
    dinv = lax.rsqrt(d0_ref[:, :1] + d1_ref[:, :1] + 1.0)
    prod = jnp.dot(h_ref[...], w_ref[...],
                   preferred_element_type=jnp.float32,
                   precision=lax.Precision.DEFAULT)
    prod = prod * dinv
    for c, o in enumerate(o_refs):
        o[...] = prod[:, c * CW:(c + 1) * CW]


def _mm_kernel(h, w, d0, d1):
    fi, fo = w.shape
    nc = fo // CW
    return pl.pallas_call(
        _mm_body,
        grid=(NBLK,),
        in_specs=[
            pl.BlockSpec((RB, fi), lambda i: (i, 0)),
            pl.BlockSpec((fi, fo), lambda i: (0, 0)),
            pl.BlockSpec((RB, _DEGW), lambda i: (i, 0)),
            pl.BlockSpec((RB, _DEGW), lambda i: (i, 0)),
        ],
        out_specs=[pl.BlockSpec((RB, CW), lambda i: (i, 0))] * nc,
        out_shape=[jax.ShapeDtypeStruct((N_PAD, CW), jnp.float32)] * nc,
    )(h, w, d0, d1)


def _fused_body(nc_in, nc_out, has_mm, *refs):
    y = refs[:nc_in]
    hsc = refs[nc_in:2 * nc_in]
    pos = 2 * nc_in
    b_ref, g_ref, be_ref, d0_ref, d1_ref = refs[pos:pos + 5]
    pos += 5
    if has_mm:
        w_ref = refs[pos]
        pos += 1
    o_refs = refs[pos:pos + nc_out]
    t_buf, acc = refs[pos + nc_out:]
    p = pl.program_id(0)
    i = pl.program_id(1)
    dinv = lax.rsqrt(d0_ref[:, :1] + d1_ref[:, :1] + 1.0)

    @pl.when(jnp.logical_and(p == 0, i == 0))
    def _():
        acc[...] = jnp.zeros_like(acc)

    @pl.when(p == 0)
    def _():
        yf = jnp.concatenate([r[...] for r in y], axis=1)
        hf = jnp.concatenate([r[...] for r in hsc], axis=1)
        t = dinv * (yf + hf) + b_ref[...]
        t_buf[pl.ds(i * RB, RB), :] = t
        acc[0:1, :] += jnp.sum(t, axis=0, keepdims=True)
        acc[1:2, :] += jnp.sum(t * t, axis=0, keepdims=True)

    @pl.when(p == 1)
    def _():
        t = t_buf[pl.ds(i * RB, RB), :]
        mu = acc[0:1, :] / N
        var = acc[1:2, :] / N - mu * mu
        hn = g_ref[...] * (t - mu) * lax.rsqrt(var + EPS) + be_ref[...]
        hn = jnp.maximum(hn, 0.0)
        if has_mm:
            prod = jnp.dot(hn, w_ref[...],
                           preferred_element_type=jnp.float32,
                           precision=lax.Precision.DEFAULT)
            prod = prod * dinv
        else:
            prod = hn
        for c, o in enumerate(o_refs):
            o[...] = prod[:, c * CW:(c + 1) * CW]


def _fused_kernel(y_chunks, hs_chunks, b, g, be, d0, d1, w):
    nc_in = len(y_chunks)
    fi = nc_in * CW
    has_mm = w is not None
    fo = w.shape[1] if has_mm else fi
    nc_out = fo // CW
    body = functools.partial(_fused_body, nc_in, nc_out, has_mm)
    row = lambda p, i: (i, 0)
    phase0_row = lambda p, i: (i * (1 - p), 0)
    const = lambda p, i: (0, 0)
    in_specs = (
        [pl.BlockSpec((RB, CW), phase0_row)] * (2 * nc_in)
        + [pl.BlockSpec((1, fi), const),
           pl.BlockSpec((1, fi), const),
           pl.BlockSpec((1, fi), const),
           pl.BlockSpec((RB, _DEGW), row),
           pl.BlockSpec((RB, _DEGW), row)]
    )
    args = list(y_chunks) + list(hs_chunks) + [
        b.reshape(1, fi), g.reshape(1, fi), be.reshape(1, fi), d0, d1]
    if has_mm:
        in_specs.append(pl.BlockSpec((fi, fo), const))
        args.append(w)
    out_rows = N_PAD if has_mm else N
    return pl.pallas_call(
        body,
        grid=(2, NBLK),
        in_specs=in_specs,
        out_specs=[pl.BlockSpec((RB, CW), lambda p, i: (i * p, 0))] * nc_out,
        out_shape=[jax.ShapeDtypeStruct((out_rows, CW), jnp.float32)] * nc_out,
        scratch_shapes=[pltpu.VMEM((N, fi), jnp.float32),
                        pltpu.VMEM((8, fi), jnp.float32)],
    )(*args)


def _final_body(p0, p1, p2, p3, cnt, o_ref):
    sums = jnp.concatenate([p0[...], p1[...], p2[...], p3[...]], axis=1)
    c = jnp.maximum(cnt[:, :1], 1.0)
    o_ref[...] = sums / c


def _final_kernel(pc, cnt):
    return pl.pallas_call(
        _final_body,
        in_specs=[pl.BlockSpec((G, CW), lambda: (0, 0))] * 4
        + [pl.BlockSpec((G, _DEGW), lambda: (0, 0))],
        out_specs=pl.BlockSpec((G, 4 * CW), lambda: (0, 0)),
        out_shape=jax.ShapeDtypeStruct((G, 4 * CW), jnp.float32),
    )(*pc, cnt)


_agg2 = _make_agg(2)
_agg4 = _make_agg(4)


def kernel(x, edge_index, batch, W1, b1, W2, b2, W3, b3, g1, be1, g2, be2, g3, be3):
    npad = E_PAD - E
    srcg = jnp.concatenate(
        [edge_index[0], jnp.full((npad,), PAD_SRC, jnp.int32)]).reshape(NEG_PAD, EG)
    dstg = jnp.concatenate(
        [edge_index[1], jnp.full((npad,), PAD_DST, jnp.int32)]).reshape(NEG_PAD, EG)
    batchg = jnp.concatenate(
        [batch, jnp.zeros((632 * _PG - N,), jnp.int32)]).reshape(632, _PG)

    d0, d1 = _deg_kernel(dstg)

    hs1 = _mm_kernel(x, W1, d0, d1)
    y1 = _agg2(srcg, dstg, *hs1)
    hs2 = _fused_kernel(y1, hs1, b1, g1, be1, d0, d1, W2)
    y2 = _agg4(srcg, dstg, *hs2)
    hs3 = _fused_kernel(y2, hs2, b2, g2, be2, d0, d1, W3)
    y3 = _agg4(srcg, dstg, *hs3)
    h = _fused_kernel(y3, hs3, b3, g3, be3, d0, d1, None)
    *pc, cnt = _pool_kernel(batchg, *h)
    return _final_kernel(pc, cnt)


# speedup vs baseline: 16.0537x; 1.0476x over previous
import functools

import jax
import jax.numpy as jnp
from jax import lax
from jax.experimental import pallas as pl
from jax.experimental.pallas import tpu as pltpu
from jax.experimental.pallas import tpu_sc as plsc

N = 50000
N_PAD = 50048
E = 800000
G = 256
EPS = 1e-5
CW = 32
EG = 128
TPS = 16
ROWS_PER_TILE = N_PAD // TPS
CP = 184
NB = 8
GPT = 392
NBLK_E = GPT // NB
NEG_PAD = GPT * TPS
E_PAD = NEG_PAD * EG
PAD_SRC = 50016
PAD_DST = 50040
RB = 1000
NBLK = N // RB

_MESH = plsc.VectorSubcoreMesh(core_axis_name="c", subcore_axis_name="s")
_SC_PARAMS = pltpu.CompilerParams(use_tc_tiling_on_sc=False)


def _zero_vmem(ref, nrows, width):
    z = jnp.zeros((16,), jnp.float32)

    def body(i, _):
        for c in range(width // 16):
            ref[i, pl.ds(c * 16, 16)] = z
        return 0

    lax.fori_loop(0, nrows, body, 0)


def _fill_ones(ref, nrows, width):
    o = jnp.ones((16,), jnp.float32)

    def body(i, _):
        for c in range(width // 16):
            ref[i, pl.ds(c * 16, 16)] = o
        return 0

    lax.fori_loop(0, nrows, body, 0)


def _make_agg(nchunks):
    npass = nchunks // 2
    out_type = [jax.ShapeDtypeStruct((N_PAD, CW), jnp.float32) for _ in range(nchunks)]
    scratch = [
        pltpu.VMEM_SHARED((N_PAD, CW), jnp.float32),
        pltpu.VMEM((3 * NB, EG), jnp.int32),
        pltpu.VMEM((3 * NB, EG), jnp.int32),
        pltpu.VMEM((5, EG, CW), jnp.float32),
        pltpu.VMEM((EG, CW), jnp.float32),
        pltpu.SemaphoreType.DMA,
        pltpu.SemaphoreType.DMA,
        pltpu.SemaphoreType.DMA,
    ]
    ncp = ROWS_PER_TILE // EG
    tail = ROWS_PER_TILE - ncp * EG

    def body(srcg, dstg, *rest):
        hs = rest[:nchunks]
        outs = rest[nchunks:2 * nchunks]
        acc, sbuf, dbuf, rows, stage, sem_i, sem_g, sem_s = rest[2 * nchunks:]
        cid = lax.axis_index("c")
        tid = lax.axis_index("s")
        rbase = tid * ROWS_PER_TILE

        def edge_loop(hs_c):
            g0 = tid * GPT
            pltpu.sync_copy(srcg.at[pl.ds(g0, NB), :], sbuf.at[pl.ds(0, NB), :])
            pltpu.sync_copy(dstg.at[pl.ds(g0, NB), :], dbuf.at[pl.ds(0, NB), :])
            pltpu.async_copy(srcg.at[pl.ds(g0 + NB, NB), :],
                             sbuf.at[pl.ds(NB, NB), :], sem_i)
            pltpu.async_copy(dstg.at[pl.ds(g0 + NB, NB), :],
                             dbuf.at[pl.ds(NB, NB), :], sem_i)
            for j in range(4):
                pltpu.async_copy(hs_c.at[sbuf.at[j]], rows.at[j], sem_g)

            def blk(b, _):
                s = lax.rem(b, 3) * NB
                for j in range(NB):
                    gg = b * NB + j
                    slot = lax.rem(gg, 5)
                    pltpu.make_async_copy(hs_c.at[sbuf.at[s + j]],
                                          rows.at[slot], sem_g).wait()

                    @pl.when(gg >= 1)
                    def _():
                        pltpu.make_async_copy(rows.at[0], acc.at[dbuf.at[0]],
                                              sem_s).wait()

                    pltpu.async_copy(rows.at[slot], acc.at[dbuf.at[s + j]],
                                     sem_s, add=True)
                    if j == 3:
                        @pl.when(b + 1 < NBLK_E)
                        def _():
                            pltpu.make_async_copy(
                                srcg.at[pl.ds(g0, NB), :],
                                sbuf.at[pl.ds(0, NB), :], sem_i).wait()
                            pltpu.make_async_copy(
                                dstg.at[pl.ds(g0, NB), :],
                                dbuf.at[pl.ds(0, NB), :], sem_i).wait()

                        @pl.when(b + 2 < NBLK_E)
                        def _():
                            gn = g0 + (b + 2) * NB
                            s2 = lax.rem(b + 2, 3) * NB
                            pltpu.async_copy(srcg.at[pl.ds(gn, NB), :],
                                             sbuf.at[pl.ds(s2, NB), :], sem_i)
                            pltpu.async_copy(dstg.at[pl.ds(gn, NB), :],
                                             dbuf.at[pl.ds(s2, NB), :], sem_i)

                    if j < NB - 4:
                        sb_row = s + j + 4
                    else:
                        sb_row = lax.rem(b + 1, 3) * NB + (j + 4 - NB)
                    gslot = lax.rem(gg + 4, 5)

                    @pl.when(gg + 4 < GPT)
                    def _(sb_row=sb_row, gslot=gslot):
                        pltpu.async_copy(hs_c.at[sbuf.at[sb_row]],
                                         rows.at[gslot], sem_g)
                return 0

            lax.fori_loop(0, NBLK_E, blk, 0)
            pltpu.make_async_copy(rows.at[0], acc.at[dbuf.at[0]], sem_s).wait()

        for p in range(npass):
            _zero_vmem(stage, EG, CW)
            for j in range(ncp):
                pltpu.sync_copy(stage, acc.at[pl.ds(rbase + j * EG, EG), :])
            pltpu.sync_copy(stage.at[pl.ds(0, tail), :],
                            acc.at[pl.ds(rbase + ncp * EG, tail), :])
            plsc.subcore_barrier()
            for sc in range(2):
                chunk = sc * npass + p

                @pl.when(cid == sc)
                def _(chunk=chunk):
                    edge_loop(hs[chunk])

            plsc.subcore_barrier()
            for sc in range(2):
                chunk = sc * npass + p

                @pl.when(cid == sc)
                def _(chunk=chunk):
                    for j in range(ncp):
                        sl = pl.ds(rbase + j * EG, EG)
                        pltpu.sync_copy(acc.at[sl, :], stage)
                        pltpu.sync_copy(stage, outs[chunk].at[sl, :])
                    sl = pl.ds(rbase + ncp * EG, tail)
                    pltpu.sync_copy(acc.at[sl, :], stage.at[pl.ds(0, tail), :])
                    pltpu.sync_copy(stage.at[pl.ds(0, tail), :],
                                    outs[chunk].at[sl, :])

            if p + 1 < npass:
                plsc.subcore_barrier()

    return pl.kernel(body, out_type=out_type, mesh=_MESH, scratch_types=scratch,
                     compiler_params=_SC_PARAMS)


_DEGW = 16
_DGRP_PER_SC = NEG_PAD // 2
_DGRP_PER_TILE = _DGRP_PER_SC // TPS
_DNB = 28


def _deg_body(dstg, out0, out1, dacc, didx, ones, zstage, sem_i, sem_s):
    cid = lax.axis_index("c")
    tid = lax.axis_index("s")
    rbase = tid * ROWS_PER_TILE
    g0 = cid * _DGRP_PER_SC + tid * _DGRP_PER_TILE

    _zero_vmem(zstage, CP, _DEGW)
    _fill_ones(ones, EG, _DEGW)
    for j in range(ROWS_PER_TILE // CP):
        pltpu.sync_copy(zstage, dacc.at[pl.ds(rbase + j * CP, CP), :])
    plsc.subcore_barrier()

    nblk = _DGRP_PER_TILE // _DNB
    pltpu.async_copy(dstg.at[pl.ds(g0, _DNB), :],
                     didx.at[pl.ds(0, _DNB), :], sem_i)

    def blk(b, _):
        s = lax.rem(b, 2) * _DNB
        pltpu.make_async_copy(dstg.at[pl.ds(g0, _DNB), :],
                              didx.at[pl.ds(0, _DNB), :], sem_i).wait()

        @pl.when(b + 1 < nblk)
        def _():
            pltpu.async_copy(dstg.at[pl.ds(g0 + (b + 1) * _DNB, _DNB), :],
                             didx.at[pl.ds(_DNB - s, _DNB), :], sem_i)

        for j in range(_DNB):
            pltpu.async_copy(ones, dacc.at[didx.at[s + j]], sem_s, add=True)
        for j in range(_DNB):
            pltpu.make_async_copy(ones, dacc.at[didx.at[s + j]], sem_s).wait()
        return 0

    lax.fori_loop(0, nblk, blk, 0)
    plsc.subcore_barrier()

    for sc, out in ((0, out0), (1, out1)):
        @pl.when(cid == sc)
        def _(out=out):
            for j in range(ROWS_PER_TILE // CP):
                sl = pl.ds(rbase + j * CP, CP)
                pltpu.sync_copy(dacc.at[sl, :], zstage)
                pltpu.sync_copy(zstage, out.at[sl, :])


_deg_kernel = pl.kernel(
    _deg_body,
    out_type=[jax.ShapeDtypeStruct((N_PAD, _DEGW), jnp.float32)] * 2,
    mesh=_MESH,
    scratch_types=[
        pltpu.VMEM_SHARED((N_PAD, _DEGW), jnp.float32),
        pltpu.VMEM((2 * _DNB, EG), jnp.int32),
        pltpu.VMEM((EG, _DEGW), jnp.float32),
        pltpu.VMEM((CP, _DEGW), jnp.float32),
        pltpu.SemaphoreType.DMA,
        pltpu.SemaphoreType.DMA,
    ],
    compiler_params=_SC_PARAMS,
)


_PG = 80
_NPG = N // _PG
_PROWS = G // TPS
_PGMAX = 40


def _pool_body(batchg, h0, h1, h2, h3, s0, s1, s2, s3, cnt_out,
               pacc, cacc, bidx, hrows, ones, pstage, cstage, sem_g, sem_c):
    hs = (h0, h1, h2, h3)
    outs = (s0, s1, s2, s3)
    cid = lax.axis_index("c")
    tid = lax.axis_index("s")
    rbase = tid * _PROWS
    gs = jnp.where(tid == 0, 0, _PGMAX + (tid - 1) * (_PGMAX - 1))
    ngrp = jnp.where(tid == 0, _PGMAX, _PGMAX - 1)

    _zero_vmem(pstage, _PROWS, CW)
    _zero_vmem(cstage, _PROWS, _DEGW)
    _fill_ones(ones, _PG, _DEGW)
    pltpu.sync_copy(batchg.at[pl.ds(gs, _PGMAX), :], bidx)

    def sum_loop(h_c, count_too):
        pltpu.async_copy(h_c.at[pl.ds(gs * _PG, _PG), :], hrows.at[0], sem_g)

        def step(g, _):
            slot = lax.rem(g, 2)
            pltpu.make_async_copy(h_c.at[pl.ds(gs * _PG, _PG), :],
                                  hrows.at[slot], sem_g).wait()

            @pl.when(g + 1 < ngrp)
            def _():
                pltpu.async_copy(h_c.at[pl.ds((gs + g + 1) * _PG, _PG), :],
                                 hrows.at[1 - slot], sem_g)

            pltpu.sync_copy(hrows.at[slot], pacc.at[bidx.at[g]], add=True)
            if count_too:
                pltpu.async_copy(ones, cacc.at[bidx.at[g]], sem_c, add=True)
            return 0

        lax.fori_loop(0, ngrp, step, 0)
        if count_too:
            def drain(g, _):
                pltpu.make_async_copy(ones, cacc.at[bidx.at[0]], sem_c).wait()
                return 0

            lax.fori_loop(0, ngrp, drain, 0)

    for p in range(2):
        pltpu.sync_copy(pstage, pacc.at[pl.ds(rbase, _PROWS), :])
        if p == 0:
            pltpu.sync_copy(cstage, cacc.at[pl.ds(rbase, _PROWS), :])
        plsc.subcore_barrier()
        for sc in range(2):
            chunk = sc * 2 + p

            @pl.when(cid == sc)
            def _(chunk=chunk, sc=sc):
                sum_loop(hs[chunk], count_too=(p == 0 and sc == 0))

        plsc.subcore_barrier()
        for sc in range(2):
            chunk = sc * 2 + p

            @pl.when(cid == sc)
            def _(chunk=chunk):
                sl = pl.ds(rbase, _PROWS)
                pltpu.sync_copy(pacc.at[sl, :], pstage)
                pltpu.sync_copy(pstage, outs[chunk].at[sl, :])
                _zero_vmem(pstage, _PROWS, CW)

        if p == 0:
            @pl.when(cid == 0)
            def _():
                sl = pl.ds(rbase, _PROWS)
                pltpu.sync_copy(cacc.at[sl, :], cstage)
                pltpu.sync_copy(cstage, cnt_out.at[sl, :])

        if p == 0:
            plsc.subcore_barrier()


_pool_kernel = pl.kernel(
    _pool_body,
    out_type=[jax.ShapeDtypeStruct((G, CW), jnp.float32)] * 4
    + [jax.ShapeDtypeStruct((G, _DEGW), jnp.float32)],
    mesh=_MESH,
    scratch_types=[
        pltpu.VMEM_SHARED((G, CW), jnp.float32),
        pltpu.VMEM_SHARED((G, _DEGW), jnp.float32),
        pltpu.VMEM((_PGMAX, _PG), jnp.int32),
        pltpu.VMEM((2, _PG, CW), jnp.float32),
        pltpu.VMEM((_PG, _DEGW), jnp.float32),
        pltpu.VMEM((_PROWS, CW), jnp.float32),
        pltpu.VMEM((_PROWS, _DEGW), jnp.float32),
        pltpu.SemaphoreType.DMA,
        pltpu.SemaphoreType.DMA,
    ],
    compiler_params=_SC_PARAMS,
)


def _mm_body(h_ref, w_ref, d0_ref, d1_ref, *o_refs):
    dinv = lax.rsqrt(d0_ref[:, :1] + d1_ref[:, :1] + 1.0)
    prod = jnp.dot(h_ref[...], w_ref[...],
                   preferred_element_type=jnp.float32,
                   precision=lax.Precision.DEFAULT)
    prod = prod * dinv
    for c, o in enumerate(o_refs):
        o[...] = prod[:, c * CW:(c + 1) * CW]


def _mm_kernel(h, w, d0, d1):
    fi, fo = w.shape
    nc = fo // CW
    return pl.pallas_call(
        _mm_body,
        grid=(NBLK,),
        in_specs=[
            pl.BlockSpec((RB, fi), lambda i: (i, 0)),
            pl.BlockSpec((fi, fo), lambda i: (0, 0)),
            pl.BlockSpec((RB, _DEGW), lambda i: (i, 0)),
            pl.BlockSpec((RB, _DEGW), lambda i: (i, 0)),
        ],
        out_specs=[pl.BlockSpec((RB, CW), lambda i: (i, 0))] * nc,
        out_shape=[jax.ShapeDtypeStruct((N_PAD, CW), jnp.float32)] * nc,
    )(h, w, d0, d1)


def _fused_body(nc_in, nc_out, has_mm, *refs):
    y = refs[:nc_in]
    hsc = refs[nc_in:2 * nc_in]
    pos = 2 * nc_in
    b_ref, g_ref, be_ref, d0_ref, d1_ref = refs[pos:pos + 5]
    pos += 5
    if has_mm:
        w_ref = refs[pos]
        pos += 1
    o_refs = refs[pos:pos + nc_out]
    t_buf, acc = refs[pos + nc_out:]
    p = pl.program_id(0)
    i = pl.program_id(1)
    dinv = lax.rsqrt(d0_ref[:, :1] + d1_ref[:, :1] + 1.0)

    @pl.when(jnp.logical_and(p == 0, i == 0))
    def _():
        acc[...] = jnp.zeros_like(acc)

    @pl.when(p == 0)
    def _():
        yf = jnp.concatenate([r[...] for r in y], axis=1)
        hf = jnp.concatenate([r[...] for r in hsc], axis=1)
        t = dinv * (yf + hf) + b_ref[...]
        t_buf[pl.ds(i * RB, RB), :] = t
        acc[0:1, :] += jnp.sum(t, axis=0, keepdims=True)
        acc[1:2, :] += jnp.sum(t * t, axis=0, keepdims=True)

    @pl.when(p == 1)
    def _():
        t = t_buf[pl.ds(i * RB, RB), :]
        mu = acc[0:1, :] / N
        var = acc[1:2, :] / N - mu * mu
        hn = g_ref[...] * (t - mu) * lax.rsqrt(var + EPS) + be_ref[...]
        hn = jnp.maximum(hn, 0.0)
        if has_mm:
            prod = jnp.dot(hn, w_ref[...],
                           preferred_element_type=jnp.float32,
                           precision=lax.Precision.DEFAULT)
            prod = prod * dinv
        else:
            prod = hn
        for c, o in enumerate(o_refs):
            o[...] = prod[:, c * CW:(c + 1) * CW]


def _fused_kernel(y_chunks, hs_chunks, b, g, be, d0, d1, w):
    nc_in = len(y_chunks)
    fi = nc_in * CW
    has_mm = w is not None
    fo = w.shape[1] if has_mm else fi
    nc_out = fo // CW
    body = functools.partial(_fused_body, nc_in, nc_out, has_mm)
    row = lambda p, i: (i, 0)
    phase0_row = lambda p, i: (i * (1 - p), 0)
    const = lambda p, i: (0, 0)
    in_specs = (
        [pl.BlockSpec((RB, CW), phase0_row)] * (2 * nc_in)
        + [pl.BlockSpec((1, fi), const),
           pl.BlockSpec((1, fi), const),
           pl.BlockSpec((1, fi), const),
           pl.BlockSpec((RB, _DEGW), row),
           pl.BlockSpec((RB, _DEGW), row)]
    )
    args = list(y_chunks) + list(hs_chunks) + [
        b.reshape(1, fi), g.reshape(1, fi), be.reshape(1, fi), d0, d1]
    if has_mm:
        in_specs.append(pl.BlockSpec((fi, fo), const))
        args.append(w)
    out_rows = N_PAD if has_mm else N
    return pl.pallas_call(
        body,
        grid=(2, NBLK),
        in_specs=in_specs,
        out_specs=[pl.BlockSpec((RB, CW), lambda p, i: (i * p, 0))] * nc_out,
        out_shape=[jax.ShapeDtypeStruct((out_rows, CW), jnp.float32)] * nc_out,
        scratch_shapes=[pltpu.VMEM((N, fi), jnp.float32),
                        pltpu.VMEM((8, fi), jnp.float32)],
    )(*args)


def _final_body(p0, p1, p2, p3, cnt, o_ref):
    sums = jnp.concatenate([p0[...], p1[...], p2[...], p3[...]], axis=1)
    c = jnp.maximum(cnt[:, :1], 1.0)
    o_ref[...] = sums / c


def _final_kernel(pc, cnt):
    return pl.pallas_call(
        _final_body,
        in_specs=[pl.BlockSpec((G, CW), lambda: (0, 0))] * 4
        + [pl.BlockSpec((G, _DEGW), lambda: (0, 0))],
        out_specs=pl.BlockSpec((G, 4 * CW), lambda: (0, 0)),
        out_shape=jax.ShapeDtypeStruct((G, 4 * CW), jnp.float32),
    )(*pc, cnt)


_agg2 = _make_agg(2)
_agg4 = _make_agg(4)


def kernel(x, edge_index, batch, W1, b1, W2, b2, W3, b3, g1, be1, g2, be2, g3, be3):
    npad = E_PAD - E
    srcg = jnp.concatenate(
        [edge_index[0], jnp.full((npad,), PAD_SRC, jnp.int32)]).reshape(NEG_PAD, EG)
    dstg = jnp.concatenate(
        [edge_index[1], jnp.full((npad,), PAD_DST, jnp.int32)]).reshape(NEG_PAD, EG)
    batchg = jnp.concatenate(
        [batch, jnp.zeros((632 * _PG - N,), jnp.int32)]).reshape(632, _PG)

    d0, d1 = _deg_kernel(dstg)

    hs1 = _mm_kernel(x, W1, d0, d1)
    y1 = _agg2(srcg, dstg, *hs1)
    hs2 = _fused_kernel(y1, hs1, b1, g1, be1, d0, d1, W2)
    y2 = _agg4(srcg, dstg, *hs2)
    hs3 = _fused_kernel(y2, hs2, b2, g2, be2, d0, d1, W3)
    y3 = _agg4(srcg, dstg, *hs3)
    h = _fused_kernel(y3, hs3, b3, g3, be3, d0, d1, None)
    *pc, cnt = _pool_kernel(batchg, *h)
    return _final_kernel(pc, cnt)


# speedup vs baseline: 17.0050x; 1.0593x over previous
import functools

import jax
import jax.numpy as jnp
from jax import lax
from jax.experimental import pallas as pl
from jax.experimental.pallas import tpu as pltpu
from jax.experimental.pallas import tpu_sc as plsc

N = 50000
N_PAD = 50048
E = 800000
G = 256
EPS = 1e-5
CW = 32
EG = 128
TPS = 16
ROWS_PER_TILE = N_PAD // TPS
CP = 184
NB = 8
GPT = 392
NBLK_E = GPT // NB
NEG_PAD = GPT * TPS
E_PAD = NEG_PAD * EG
PAD_SRC = 50016
PAD_DST = 50040
RB = 2000
NBLK = N // RB

_MESH = plsc.VectorSubcoreMesh(core_axis_name="c", subcore_axis_name="s")
_SC_PARAMS = pltpu.CompilerParams(use_tc_tiling_on_sc=False)


def _zero_vmem(ref, nrows, width):
    z = jnp.zeros((16,), jnp.float32)

    def body(i, _):
        for c in range(width // 16):
            ref[i, pl.ds(c * 16, 16)] = z
        return 0

    lax.fori_loop(0, nrows, body, 0)


def _fill_ones(ref, nrows, width):
    o = jnp.ones((16,), jnp.float32)

    def body(i, _):
        for c in range(width // 16):
            ref[i, pl.ds(c * 16, 16)] = o
        return 0

    lax.fori_loop(0, nrows, body, 0)


def _make_agg(nchunks):
    npass = nchunks // 2
    out_type = [jax.ShapeDtypeStruct((N_PAD, CW), jnp.float32) for _ in range(nchunks)]
    scratch = [
        pltpu.VMEM_SHARED((N_PAD, CW), jnp.float32),
        pltpu.VMEM((3 * NB, EG), jnp.int32),
        pltpu.VMEM((3 * NB, EG), jnp.int32),
        pltpu.VMEM((5, EG, CW), jnp.float32),
        pltpu.VMEM((EG, CW), jnp.float32),
        pltpu.SemaphoreType.DMA,
        pltpu.SemaphoreType.DMA,
        pltpu.SemaphoreType.DMA,
    ]
    ncp = ROWS_PER_TILE // EG
    tail = ROWS_PER_TILE - ncp * EG

    def body(srcg, dstg, *rest):
        hs = rest[:nchunks]
        outs = rest[nchunks:2 * nchunks]
        acc, sbuf, dbuf, rows, stage, sem_i, sem_g, sem_s = rest[2 * nchunks:]
        cid = lax.axis_index("c")
        tid = lax.axis_index("s")
        rbase = tid * ROWS_PER_TILE

        def edge_loop(hs_c):
            g0 = tid * GPT
            pltpu.sync_copy(srcg.at[pl.ds(g0, NB), :], sbuf.at[pl.ds(0, NB), :])
            pltpu.sync_copy(dstg.at[pl.ds(g0, NB), :], dbuf.at[pl.ds(0, NB), :])
            pltpu.async_copy(srcg.at[pl.ds(g0 + NB, NB), :],
                             sbuf.at[pl.ds(NB, NB), :], sem_i)
            pltpu.async_copy(dstg.at[pl.ds(g0 + NB, NB), :],
                             dbuf.at[pl.ds(NB, NB), :], sem_i)
            for j in range(4):
                pltpu.async_copy(hs_c.at[sbuf.at[j]], rows.at[j], sem_g)

            def blk(b, _):
                s = lax.rem(b, 3) * NB
                for j in range(NB):
                    gg = b * NB + j
                    slot = lax.rem(gg, 5)
                    pltpu.make_async_copy(hs_c.at[sbuf.at[s + j]],
                                          rows.at[slot], sem_g).wait()

                    @pl.when(gg >= 1)
                    def _():
                        pltpu.make_async_copy(rows.at[0], acc.at[dbuf.at[0]],
                                              sem_s).wait()

                    pltpu.async_copy(rows.at[slot], acc.at[dbuf.at[s + j]],
                                     sem_s, add=True)
                    if j == 3:
                        @pl.when(b + 1 < NBLK_E)
                        def _():
                            pltpu.make_async_copy(
                                srcg.at[pl.ds(g0, NB), :],
                                sbuf.at[pl.ds(0, NB), :], sem_i).wait()
                            pltpu.make_async_copy(
                                dstg.at[pl.ds(g0, NB), :],
                                dbuf.at[pl.ds(0, NB), :], sem_i).wait()

                        @pl.when(b + 2 < NBLK_E)
                        def _():
                            gn = g0 + (b + 2) * NB
                            s2 = lax.rem(b + 2, 3) * NB
                            pltpu.async_copy(srcg.at[pl.ds(gn, NB), :],
                                             sbuf.at[pl.ds(s2, NB), :], sem_i)
                            pltpu.async_copy(dstg.at[pl.ds(gn, NB), :],
                                             dbuf.at[pl.ds(s2, NB), :], sem_i)

                    if j < NB - 4:
                        sb_row = s + j + 4
                    else:
                        sb_row = lax.rem(b + 1, 3) * NB + (j + 4 - NB)
                    gslot = lax.rem(gg + 4, 5)

                    @pl.when(gg + 4 < GPT)
                    def _(sb_row=sb_row, gslot=gslot):
                        pltpu.async_copy(hs_c.at[sbuf.at[sb_row]],
                                         rows.at[gslot], sem_g)
                return 0

            lax.fori_loop(0, NBLK_E, blk, 0)
            pltpu.make_async_copy(rows.at[0], acc.at[dbuf.at[0]], sem_s).wait()

        for p in range(npass):
            _zero_vmem(stage, EG, CW)
            for j in range(ncp):
                pltpu.sync_copy(stage, acc.at[pl.ds(rbase + j * EG, EG), :])
            pltpu.sync_copy(stage.at[pl.ds(0, tail), :],
                            acc.at[pl.ds(rbase + ncp * EG, tail), :])
            plsc.subcore_barrier()
            for sc in range(2):
                chunk = sc * npass + p

                @pl.when(cid == sc)
                def _(chunk=chunk):
                    edge_loop(hs[chunk])

            plsc.subcore_barrier()
            for sc in range(2):
                chunk = sc * npass + p

                @pl.when(cid == sc)
                def _(chunk=chunk):
                    for j in range(ncp):
                        sl = pl.ds(rbase + j * EG, EG)
                        pltpu.sync_copy(acc.at[sl, :], stage)
                        pltpu.sync_copy(stage, outs[chunk].at[sl, :])
                    sl = pl.ds(rbase + ncp * EG, tail)
                    pltpu.sync_copy(acc.at[sl, :], stage.at[pl.ds(0, tail), :])
                    pltpu.sync_copy(stage.at[pl.ds(0, tail), :],
                                    outs[chunk].at[sl, :])

            if p + 1 < npass:
                plsc.subcore_barrier()

    return pl.kernel(body, out_type=out_type, mesh=_MESH, scratch_types=scratch,
                     compiler_params=_SC_PARAMS)


_DEGW = 16
_DGRP_PER_SC = NEG_PAD // 2
_DGRP_PER_TILE = _DGRP_PER_SC // TPS
_DNB = 28


def _deg_body(dstg, out0, out1, dacc, didx, ones, zstage, sem_i, sem_s):
    cid = lax.axis_index("c")
    tid = lax.axis_index("s")
    rbase = tid * ROWS_PER_TILE
    g0 = cid * _DGRP_PER_SC + tid * _DGRP_PER_TILE

    _zero_vmem(zstage, CP, _DEGW)
    _fill_ones(ones, EG, _DEGW)
    for j in range(ROWS_PER_TILE // CP):
        pltpu.sync_copy(zstage, dacc.at[pl.ds(rbase + j * CP, CP), :])
    plsc.subcore_barrier()

    nblk = _DGRP_PER_TILE // _DNB
    pltpu.async_copy(dstg.at[pl.ds(g0, _DNB), :],
                     didx.at[pl.ds(0, _DNB), :], sem_i)

    def blk(b, _):
        s = lax.rem(b, 2) * _DNB
        pltpu.make_async_copy(dstg.at[pl.ds(g0, _DNB), :],
                              didx.at[pl.ds(0, _DNB), :], sem_i).wait()

        @pl.when(b + 1 < nblk)
        def _():
            pltpu.async_copy(dstg.at[pl.ds(g0 + (b + 1) * _DNB, _DNB), :],
                             didx.at[pl.ds(_DNB - s, _DNB), :], sem_i)

        for j in range(_DNB):
            pltpu.async_copy(ones, dacc.at[didx.at[s + j]], sem_s, add=True)
        for j in range(_DNB):
            pltpu.make_async_copy(ones, dacc.at[didx.at[s + j]], sem_s).wait()
        return 0

    lax.fori_loop(0, nblk, blk, 0)
    plsc.subcore_barrier()

    for sc, out in ((0, out0), (1, out1)):
        @pl.when(cid == sc)
        def _(out=out):
            for j in range(ROWS_PER_TILE // CP):
                sl = pl.ds(rbase + j * CP, CP)
                pltpu.sync_copy(dacc.at[sl, :], zstage)
                pltpu.sync_copy(zstage, out.at[sl, :])


_deg_kernel = pl.kernel(
    _deg_body,
    out_type=[jax.ShapeDtypeStruct((N_PAD, _DEGW), jnp.float32)] * 2,
    mesh=_MESH,
    scratch_types=[
        pltpu.VMEM_SHARED((N_PAD, _DEGW), jnp.float32),
        pltpu.VMEM((2 * _DNB, EG), jnp.int32),
        pltpu.VMEM((EG, _DEGW), jnp.float32),
        pltpu.VMEM((CP, _DEGW), jnp.float32),
        pltpu.SemaphoreType.DMA,
        pltpu.SemaphoreType.DMA,
    ],
    compiler_params=_SC_PARAMS,
)


_PG = 80
_NPG = N // _PG
_PROWS = G // TPS
_PGMAX = 40


def _pool_body(batchg, h0, h1, h2, h3, s0, s1, s2, s3, cnt_out,
               pacc, cacc, bidx, hrows, ones, pstage, cstage, sem_g, sem_c):
    hs = (h0, h1, h2, h3)
    outs = (s0, s1, s2, s3)
    cid = lax.axis_index("c")
    tid = lax.axis_index("s")
    rbase = tid * _PROWS
    gs = jnp.where(tid == 0, 0, _PGMAX + (tid - 1) * (_PGMAX - 1))
    ngrp = jnp.where(tid == 0, _PGMAX, _PGMAX - 1)

    _zero_vmem(pstage, _PROWS, CW)
    _zero_vmem(cstage, _PROWS, _DEGW)
    _fill_ones(ones, _PG, _DEGW)
    pltpu.sync_copy(batchg.at[pl.ds(gs, _PGMAX), :], bidx)

    def sum_loop(h_c, count_too):
        pltpu.async_copy(h_c.at[pl.ds(gs * _PG, _PG), :], hrows.at[0], sem_g)

        def step(g, _):
            slot = lax.rem(g, 2)
            pltpu.make_async_copy(h_c.at[pl.ds(gs * _PG, _PG), :],
                                  hrows.at[slot], sem_g).wait()

            @pl.when(g + 1 < ngrp)
            def _():
                pltpu.async_copy(h_c.at[pl.ds((gs + g + 1) * _PG, _PG), :],
                                 hrows.at[1 - slot], sem_g)

            pltpu.sync_copy(hrows.at[slot], pacc.at[bidx.at[g]], add=True)
            if count_too:
                pltpu.async_copy(ones, cacc.at[bidx.at[g]], sem_c, add=True)
            return 0

        lax.fori_loop(0, ngrp, step, 0)
        if count_too:
            def drain(g, _):
                pltpu.make_async_copy(ones, cacc.at[bidx.at[0]], sem_c).wait()
                return 0

            lax.fori_loop(0, ngrp, drain, 0)

    for p in range(2):
        pltpu.sync_copy(pstage, pacc.at[pl.ds(rbase, _PROWS), :])
        if p == 0:
            pltpu.sync_copy(cstage, cacc.at[pl.ds(rbase, _PROWS), :])
        plsc.subcore_barrier()
        for sc in range(2):
            chunk = sc * 2 + p

            @pl.when(cid == sc)
            def _(chunk=chunk, sc=sc):
                sum_loop(hs[chunk], count_too=(p == 0 and sc == 0))

        plsc.subcore_barrier()
        for sc in range(2):
            chunk = sc * 2 + p

            @pl.when(cid == sc)
            def _(chunk=chunk):
                sl = pl.ds(rbase, _PROWS)
                pltpu.sync_copy(pacc.at[sl, :], pstage)
                pltpu.sync_copy(pstage, outs[chunk].at[sl, :])
                _zero_vmem(pstage, _PROWS, CW)

        if p == 0:
            @pl.when(cid == 0)
            def _():
                sl = pl.ds(rbase, _PROWS)
                pltpu.sync_copy(cacc.at[sl, :], cstage)
                pltpu.sync_copy(cstage, cnt_out.at[sl, :])

        if p == 0:
            plsc.subcore_barrier()


_pool_kernel = pl.kernel(
    _pool_body,
    out_type=[jax.ShapeDtypeStruct((G, CW), jnp.float32)] * 4
    + [jax.ShapeDtypeStruct((G, _DEGW), jnp.float32)],
    mesh=_MESH,
    scratch_types=[
        pltpu.VMEM_SHARED((G, CW), jnp.float32),
        pltpu.VMEM_SHARED((G, _DEGW), jnp.float32),
        pltpu.VMEM((_PGMAX, _PG), jnp.int32),
        pltpu.VMEM((2, _PG, CW), jnp.float32),
        pltpu.VMEM((_PG, _DEGW), jnp.float32),
        pltpu.VMEM((_PROWS, CW), jnp.float32),
        pltpu.VMEM((_PROWS, _DEGW), jnp.float32),
        pltpu.SemaphoreType.DMA,
        pltpu.SemaphoreType.DMA,
    ],
    compiler_params=_SC_PARAMS,
)


def _mm_body(h_ref, w_ref, d0_ref, d1_ref, *o_refs):
    dinv = lax.rsqrt(d0_ref[:, :1] + d1_ref[:, :1] + 1.0)
    prod = jnp.dot(h_ref[...], w_ref[...],
                   preferred_element_type=jnp.float32,
                   precision=lax.Precision.DEFAULT)
    prod = prod * dinv
    for c, o in enumerate(o_refs):
        o[...] = prod[:, c * CW:(c + 1) * CW]


def _mm_kernel(h, w, d0, d1):
    fi, fo = w.shape
    nc = fo // CW
    return pl.pallas_call(
        _mm_body,
        grid=(NBLK,),
        in_specs=[
            pl.BlockSpec((RB, fi), lambda i: (i, 0)),
            pl.BlockSpec((fi, fo), lambda i: (0, 0)),
            pl.BlockSpec((RB, _DEGW), lambda i: (i, 0)),
            pl.BlockSpec((RB, _DEGW), lambda i: (i, 0)),
        ],
        out_specs=[pl.BlockSpec((RB, CW), lambda i: (i, 0))] * nc,
        out_shape=[jax.ShapeDtypeStruct((N_PAD, CW), jnp.float32)] * nc,
    )(h, w, d0, d1)


def _fused_body(nc_in, nc_out, has_mm, *refs):
    y = refs[:nc_in]
    hsc = refs[nc_in:2 * nc_in]
    pos = 2 * nc_in
    b_ref, g_ref, be_ref, d0_ref, d1_ref = refs[pos:pos + 5]
    pos += 5
    if has_mm:
        w_ref = refs[pos]
        pos += 1
    o_refs = refs[pos:pos + nc_out]
    t_buf, acc = refs[pos + nc_out:]
    p = pl.program_id(0)
    i = pl.program_id(1)
    dinv = lax.rsqrt(d0_ref[:, :1] + d1_ref[:, :1] + 1.0)

    @pl.when(jnp.logical_and(p == 0, i == 0))
    def _():
        acc[...] = jnp.zeros_like(acc)

    @pl.when(p == 0)
    def _():
        yf = jnp.concatenate([r[...] for r in y], axis=1)
        hf = jnp.concatenate([r[...] for r in hsc], axis=1)
        t = dinv * (yf + hf) + b_ref[...]
        t_buf[pl.ds(i * RB, RB), :] = t
        acc[0:1, :] += jnp.sum(t, axis=0, keepdims=True)
        acc[1:2, :] += jnp.sum(t * t, axis=0, keepdims=True)

    @pl.when(p == 1)
    def _():
        t = t_buf[pl.ds(i * RB, RB), :]
        mu = acc[0:1, :] / N
        var = acc[1:2, :] / N - mu * mu
        hn = g_ref[...] * (t - mu) * lax.rsqrt(var + EPS) + be_ref[...]
        hn = jnp.maximum(hn, 0.0)
        if has_mm:
            prod = jnp.dot(hn, w_ref[...],
                           preferred_element_type=jnp.float32,
                           precision=lax.Precision.DEFAULT)
            prod = prod * dinv
        else:
            prod = hn
        for c, o in enumerate(o_refs):
            o[...] = prod[:, c * CW:(c + 1) * CW]


def _fused_kernel(y_chunks, hs_chunks, b, g, be, d0, d1, w):
    nc_in = len(y_chunks)
    fi = nc_in * CW
    has_mm = w is not None
    fo = w.shape[1] if has_mm else fi
    nc_out = fo // CW
    body = functools.partial(_fused_body, nc_in, nc_out, has_mm)
    row = lambda p, i: (i, 0)
    phase0_row = lambda p, i: (i * (1 - p), 0)
    const = lambda p, i: (0, 0)
    in_specs = (
        [pl.BlockSpec((RB, CW), phase0_row)] * (2 * nc_in)
        + [pl.BlockSpec((1, fi), const),
           pl.BlockSpec((1, fi), const),
           pl.BlockSpec((1, fi), const),
           pl.BlockSpec((RB, _DEGW), row),
           pl.BlockSpec((RB, _DEGW), row)]
    )
    args = list(y_chunks) + list(hs_chunks) + [
        b.reshape(1, fi), g.reshape(1, fi), be.reshape(1, fi), d0, d1]
    if has_mm:
        in_specs.append(pl.BlockSpec((fi, fo), const))
        args.append(w)
    out_rows = N_PAD if has_mm else N
    return pl.pallas_call(
        body,
        grid=(2, NBLK),
        in_specs=in_specs,
        out_specs=[pl.BlockSpec((RB, CW), lambda p, i: (i * p, 0))] * nc_out,
        out_shape=[jax.ShapeDtypeStruct((out_rows, CW), jnp.float32)] * nc_out,
        scratch_shapes=[pltpu.VMEM((N, fi), jnp.float32),
                        pltpu.VMEM((8, fi), jnp.float32)],
    )(*args)


def _final_body(p0, p1, p2, p3, cnt, o_ref):
    sums = jnp.concatenate([p0[...], p1[...], p2[...], p3[...]], axis=1)
    c = jnp.maximum(cnt[:, :1], 1.0)
    o_ref[...] = sums / c


def _final_kernel(pc, cnt):
    return pl.pallas_call(
        _final_body,
        in_specs=[pl.BlockSpec((G, CW), lambda: (0, 0))] * 4
        + [pl.BlockSpec((G, _DEGW), lambda: (0, 0))],
        out_specs=pl.BlockSpec((G, 4 * CW), lambda: (0, 0)),
        out_shape=jax.ShapeDtypeStruct((G, 4 * CW), jnp.float32),
    )(*pc, cnt)


_agg2 = _make_agg(2)
_agg4 = _make_agg(4)


def kernel(x, edge_index, batch, W1, b1, W2, b2, W3, b3, g1, be1, g2, be2, g3, be3):
    npad = E_PAD - E
    srcg = jnp.concatenate(
        [edge_index[0], jnp.full((npad,), PAD_SRC, jnp.int32)]).reshape(NEG_PAD, EG)
    dstg = jnp.concatenate(
        [edge_index[1], jnp.full((npad,), PAD_DST, jnp.int32)]).reshape(NEG_PAD, EG)
    batchg = jnp.concatenate(
        [batch, jnp.zeros((632 * _PG - N,), jnp.int32)]).reshape(632, _PG)

    d0, d1 = _deg_kernel(dstg)

    hs1 = _mm_kernel(x, W1, d0, d1)
    y1 = _agg2(srcg, dstg, *hs1)
    hs2 = _fused_kernel(y1, hs1, b1, g1, be1, d0, d1, W2)
    y2 = _agg4(srcg, dstg, *hs2)
    hs3 = _fused_kernel(y2, hs2, b2, g2, be2, d0, d1, W3)
    y3 = _agg4(srcg, dstg, *hs3)
    h = _fused_kernel(y3, hs3, b3, g3, be3, d0, d1, None)
    *pc, cnt = _pool_kernel(batchg, *h)
    return _final_kernel(pc, cnt)
